# Initial kernel scaffold; baseline (speedup 1.0000x reference)
import jax
import jax.numpy as jnp
from jax.experimental import pallas as pl


def kernel(x, edge_index, W_w, W_b, U_w, U_b, V_w, V_b, D_w, D_b, E_w, E_b):
    raise NotImplementedError("write your pallas kernel here")


# speedup vs baseline: 2.7015x; 2.7015x over previous
import functools

import jax
import jax.numpy as jnp
from jax import lax
from jax.experimental import pallas as pl
from jax.experimental.pallas import tpu as pltpu
from jax.experimental.pallas import tpu_sc as plsc

N = 10000
D = 256
Q = D // 4
SENT = N
NSUB = 16
NCORE = 2
C = 96
HS = 10240
HB = HS // NSUB
ACC_R = N + NSUB
WB = 2000
BS = 8
E_IN = 160000
EPT1 = E_IN // NSUB
ET = 172032
TPT = ET // NSUB
NCH = TPT // C
NBLK = NCH // BS
PAD_OFF = E_IN + N
ROWS_A = 632


def _rsqrt_newton(xx):
    bits = plsc.bitcast(xx, jnp.int32)
    y = plsc.bitcast(
        jnp.int32(0x5F3759DF) - lax.shift_right_logical(bits, 1),
        jnp.float32)
    for _ in range(3):
        y = y * (1.5 - 0.5 * xx * y * y)
    return y


def _tc_body(x_ref, ut_ref, ub_ref, wcat_ref, bcat_ref, wt_ref, wb_ref,
             stab_ref, dtab_ref):
    xb = x_ref[...]
    h = jnp.dot(xb, ut_ref[...], preferred_element_type=jnp.float32)
    h = h + ub_ref[...]
    y = jnp.dot(h, wcat_ref[...], preferred_element_type=jnp.float32)
    y = y + bcat_ref[...]
    xw = jnp.dot(xb, wt_ref[...], preferred_element_type=jnp.float32)
    xw = xw + wb_ref[...]
    a = y[:, 0:D]
    b = y[:, D:2 * D]
    hd = y[:, 2 * D:3 * D]
    he = y[:, 3 * D:4 * D]
    stab_ref[...] = jnp.stack(
        [jnp.concatenate([a[:, q * Q:(q + 1) * Q],
                          hd[:, q * Q:(q + 1) * Q]], axis=1)
         for q in range(4)], axis=0)
    dtab_ref[...] = jnp.stack(
        [jnp.concatenate([b[:, q * Q:(q + 1) * Q],
                          he[:, q * Q:(q + 1) * Q],
                          xw[:, q * Q:(q + 1) * Q]], axis=1)
         for q in range(4)], axis=0)


def _tc_tables(x, ut, ub, wcat, bcat, wt, wb):
    nb = 10
    blk = N // nb
    return pl.pallas_call(
        _tc_body,
        grid=(nb,),
        in_specs=[
            pl.BlockSpec((blk, D), lambda i: (i, 0)),
            pl.BlockSpec((D, D), lambda i: (0, 0)),
            pl.BlockSpec((1, D), lambda i: (0, 0)),
            pl.BlockSpec((D, 4 * D), lambda i: (0, 0)),
            pl.BlockSpec((1, 4 * D), lambda i: (0, 0)),
            pl.BlockSpec((D, D), lambda i: (0, 0)),
            pl.BlockSpec((1, D), lambda i: (0, 0)),
        ],
        out_specs=[
            pl.BlockSpec((4, blk, 2 * Q), lambda i: (0, i, 0)),
            pl.BlockSpec((4, blk, 3 * Q), lambda i: (0, i, 0)),
        ],
        out_shape=[
            jax.ShapeDtypeStruct((4, N, 2 * Q), jnp.float32),
            jax.ShapeDtypeStruct((4, N, 3 * Q), jnp.float32),
        ],
    )(x, ut, ub, wcat, bcat, wt, wb)


def _sca_body(src_hbm, dst_hbm,
              se_hbm, mse_hbm, mde_hbm, s_hbm,
              srcT, dstT, histL, wA, wC, ssum, tmpv,
              staging, histG):
    tid = lax.axis_index("s")
    cid = lax.axis_index("c")
    zero16 = jnp.zeros((16,), jnp.float32)

    base = tid * EPT1
    pltpu.sync_copy(src_hbm.at[pl.ds(base, EPT1)], srcT)
    pltpu.sync_copy(dst_hbm.at[pl.ds(base, EPT1)], dstT)

    def _zh(i, _):
        histL[pl.ds(i * 16, 16)] = zero16
        return 0
    lax.fori_loop(0, HS // 16, _zh, 0)

    def _zs(i, _):
        ssum[pl.ds(i * 16, 16)] = zero16
        return 0
    lax.fori_loop(0, HB // 16, _zs, 0)

    ones16 = jnp.ones((16,), jnp.float32)

    def _blk_a(bi, _):
        def _grp(k, _):
            j = bi * WB + k * 16
            sv = srcT[pl.ds(j, 16)]
            dv = dstT[pl.ds(j, 16)]
            m = sv != dv
            se = jnp.where(m, sv, SENT)
            plsc.addupdate_scatter(histL, [se], ones16)
            wA[pl.ds(k * 16, 16)] = se
            return 0
        lax.fori_loop(0, WB // 16, _grp, 0)

        @pl.when(cid == 0)
        def _():
            pltpu.sync_copy(wA, se_hbm.at[pl.ds(base + bi * WB, WB)])

        def _grp2(k, _):
            j = bi * WB + k * 16
            sv = srcT[pl.ds(j, 16)]
            dv = dstT[pl.ds(j, 16)]
            m = sv != dv
            wA[pl.ds(k * 16, 16)] = jnp.minimum(
                jnp.where(m, sv, SENT), N - 1)
            return 0
        lax.fori_loop(0, WB // 16, _grp2, 0)

        @pl.when(cid == 0)
        def _():
            pltpu.sync_copy(wA, mse_hbm.at[pl.ds(base + bi * WB, WB)])

        def _grp3(k, _):
            j = bi * WB + k * 16
            sv = srcT[pl.ds(j, 16)]
            dv = dstT[pl.ds(j, 16)]
            m = sv != dv
            wA[pl.ds(k * 16, 16)] = jnp.minimum(
                jnp.where(m, dv, SENT), N - 1)
            return 0
        lax.fori_loop(0, WB // 16, _grp3, 0)

        @pl.when(cid == 0)
        def _():
            pltpu.sync_copy(wA, mde_hbm.at[pl.ds(base + bi * WB, WB)])
        return 0
    lax.fori_loop(0, EPT1 // WB, _blk_a, 0)

    pltpu.sync_copy(histL, staging.at[tid])
    plsc.subcore_barrier()

    def _red(j, _):
        pltpu.sync_copy(staging.at[j, pl.ds(tid * HB, HB)], tmpv)

        def _acc(v, _):
            ssum[pl.ds(v * 16, 16)] = (ssum[pl.ds(v * 16, 16)]
                                       + tmpv[pl.ds(v * 16, 16)])
            return 0
        lax.fori_loop(0, HB // 16, _acc, 0)
        return 0
    lax.fori_loop(0, NSUB, _red, 0)
    pltpu.sync_copy(ssum, histG.at[pl.ds(tid * HB, HB)])
    plsc.subcore_barrier()

    pltpu.sync_copy(histG, histL)

    def _rsq(i, _):
        h = histL[pl.ds(i * 16, 16)]
        idx = lax.iota(jnp.int32, 16) + i * 16
        valid = idx < N
        deg = h + jnp.where(valid, 1.0, 0.0)
        y = _rsqrt_newton(jnp.maximum(deg, 1.0))
        histL[pl.ds(i * 16, 16)] = jnp.where(valid, y, 0.0)
        return 0
    lax.fori_loop(0, HS // 16, _rsq, 0)

    def _blk_b(bi, _):
        def _grp(k, _):
            j = bi * WB + k * 16
            sv = srcT[pl.ds(j, 16)]
            dv = dstT[pl.ds(j, 16)]
            m = sv != dv
            se = jnp.where(m, sv, SENT)
            de = jnp.where(m, dv, SENT)
            rs = plsc.load_gather(histL, [se])
            rd = plsc.load_gather(histL, [de])
            wC[pl.ds(k * 16, 16)] = rs * rd
            return 0
        lax.fori_loop(0, WB // 16, _grp, 0)

        @pl.when(cid == 0)
        def _():
            pltpu.sync_copy(wC, s_hbm.at[pl.ds(base + bi * WB, WB)])
        return 0
    lax.fori_loop(0, EPT1 // WB, _blk_b, 0)

    @pl.when(cid == 0)
    def _():
        nrows = jnp.where(tid < NSUB - 1, 0, 0)
        del nrows

        def _self(nrows):
            nch = -(-nrows // 16)

            def _g(k, _):
                node = tid * ROWS_A + k * 16 + lax.iota(jnp.int32, 16)
                node = jnp.minimum(node, N - 1)
                wA[pl.ds(k * 16, 16)] = node
                r = plsc.load_gather(histL, [node])
                wC[pl.ds(k * 16, 16)] = r * r
                return 0
            lax.fori_loop(0, nch, _g, 0)
            off = E_IN + tid * ROWS_A
            pltpu.sync_copy(wA.at[pl.ds(0, nrows)],
                            se_hbm.at[pl.ds(off, nrows)])
            pltpu.sync_copy(wA.at[pl.ds(0, nrows)],
                            mse_hbm.at[pl.ds(off, nrows)])
            pltpu.sync_copy(wA.at[pl.ds(0, nrows)],
                            mde_hbm.at[pl.ds(off, nrows)])
            pltpu.sync_copy(wC.at[pl.ds(0, nrows)],
                            s_hbm.at[pl.ds(off, nrows)])

        @pl.when(tid < NSUB - 1)
        def _():
            _self(ROWS_A)

        @pl.when(tid == NSUB - 1)
        def _():
            _self(N - (NSUB - 1) * ROWS_A)
            npad = ET - PAD_OFF

            def _gp(k, _):
                wA[pl.ds(k * 16, 16)] = jnp.full((16,), SENT, jnp.int32)
                wC[pl.ds(k * 16, 16)] = jnp.zeros((16,), jnp.float32)
                return 0
            lax.fori_loop(0, WB // 16, _gp, 0)
            done = 0
            while done < npad:
                n = min(WB, npad - done)
                pltpu.sync_copy(wA.at[pl.ds(0, n)],
                                se_hbm.at[pl.ds(PAD_OFF + done, n)])
                pltpu.sync_copy(wC.at[pl.ds(0, n)],
                                s_hbm.at[pl.ds(PAD_OFF + done, n)])
                done += n

            def _gq(k, _):
                wA[pl.ds(k * 16, 16)] = jnp.full((16,), N - 1, jnp.int32)
                return 0
            lax.fori_loop(0, WB // 16, _gq, 0)
            done = 0
            while done < npad:
                n = min(WB, npad - done)
                pltpu.sync_copy(wA.at[pl.ds(0, n)],
                                mse_hbm.at[pl.ds(PAD_OFF + done, n)])
                pltpu.sync_copy(wA.at[pl.ds(0, n)],
                                mde_hbm.at[pl.ds(PAD_OFF + done, n)])
                done += n


def _sc_scalers(src_p, dst_p):
    mesh = plsc.VectorSubcoreMesh(core_axis_name="c", subcore_axis_name="s",
                                  num_cores=NCORE, num_subcores=NSUB)
    kern = pl.kernel(
        _sca_body,
        out_type=[
            jax.ShapeDtypeStruct((ET,), jnp.int32),
            jax.ShapeDtypeStruct((ET,), jnp.int32),
            jax.ShapeDtypeStruct((ET,), jnp.int32),
            jax.ShapeDtypeStruct((ET,), jnp.float32),
        ],
        mesh=mesh,
        compiler_params=pltpu.CompilerParams(use_tc_tiling_on_sc=False,
                                             needs_layout_passes=False),
        scratch_types=[
            pltpu.VMEM((EPT1,), jnp.int32),
            pltpu.VMEM((EPT1,), jnp.int32),
            pltpu.VMEM((HS,), jnp.float32),
            pltpu.VMEM((WB,), jnp.int32),
            pltpu.VMEM((WB,), jnp.float32),
            pltpu.VMEM((HB,), jnp.float32),
            pltpu.VMEM((HB,), jnp.float32),
            pltpu.VMEM_SHARED((NSUB, HS), jnp.float32),
            pltpu.VMEM_SHARED((HS,), jnp.float32),
        ],
    )
    return kern(src_p, dst_p)


def _scb_body(se_hbm, mse_hbm, mde_hbm, s_hbm, stab_hbm, dtab_hbm,
              out_hbm,
              seS, mseS, mdeS, sS,
              srows, drows, msgB, idxS, idxD, sidx,
              acc, semG, semW):
    tid = lax.axis_index("s")
    cid = lax.axis_index("c")
    zero16 = jnp.zeros((16,), jnp.float32)
    base = tid * TPT

    def _zero_msg():
        def _zm(i, _):
            r = i // (Q // 16)
            c = (i % (Q // 16)) * 16
            msgB[r, pl.ds(c, 16)] = zero16
            return 0
        lax.fori_loop(0, C * (Q // 16), _zm, 0)

    def _zero_acc():
        def _za(nrows):
            off = 0
            while off < nrows:
                n = min(C, nrows - off)
                pltpu.sync_copy(msgB.at[pl.ds(0, n)],
                                acc.at[pl.ds(tid * ROWS_A + off, n)])
                off += n

        @pl.when(tid < NSUB - 1)
        def _():
            _za(ROWS_A)

        @pl.when(tid == NSUB - 1)
        def _():
            _za(ACC_R - (NSUB - 1) * ROWS_A)

    def _run_pass(p):
        qoff = (cid * 2 + p) * N

        def _blk(bi, _):
            boff = base + bi * (BS * C)
            pltpu.sync_copy(se_hbm.at[pl.ds(boff, BS * C)], seS)
            pltpu.sync_copy(mse_hbm.at[pl.ds(boff, BS * C)], mseS)
            pltpu.sync_copy(mde_hbm.at[pl.ds(boff, BS * C)], mdeS)
            pltpu.sync_copy(s_hbm.at[pl.ds(boff, BS * C)], sS)

            def _chunk(cj, _):
                o = cj * C

                def _fill(k, _):
                    j = o + k * 16
                    idxS[pl.ds(k * 16, 16)] = qoff + mseS[pl.ds(j, 16)]
                    idxD[pl.ds(k * 16, 16)] = qoff + mdeS[pl.ds(j, 16)]
                    sidx[pl.ds(k * 16, 16)] = seS[pl.ds(j, 16)]
                    return 0
                lax.fori_loop(0, C // 16, _fill, 0)

                gs = pltpu.async_copy(stab_hbm.at[idxS], srows, semG)
                gd = pltpu.async_copy(dtab_hbm.at[idxD], drows, semG)
                gs.wait()
                gd.wait()

                def _pe(e, _):
                    s = plsc.load_gather(
                        sS, [jnp.full((16,), o + e, jnp.int32)])
                    for v in range(Q // 16):
                        a = srows[e, pl.ds(v * 16, 16)]
                        hd = srows[e, pl.ds(Q + v * 16, 16)]
                        b = drows[e, pl.ds(v * 16, 16)]
                        he = drows[e, pl.ds(Q + v * 16, 16)]
                        xw = drows[e, pl.ds(2 * Q + v * 16, 16)]
                        g = jnp.maximum(a + b, 0.0) + hd + he
                        msgB[e, pl.ds(v * 16, 16)] = xw * (s * g)
                    return 0
                lax.fori_loop(0, C, _pe, 0)
                pltpu.async_copy(msgB, acc.at[sidx], semW, add=True).wait()
                return 0
            lax.fori_loop(0, BS, _chunk, 0)
            return 0
        lax.fori_loop(0, NBLK, _blk, 0)
        plsc.subcore_barrier()

        def _co(nrows):
            off = 0
            while off < nrows:
                n = min(C, nrows - off)
                pltpu.sync_copy(acc.at[pl.ds(tid * ROWS_A + off, n)],
                                msgB.at[pl.ds(0, n)])
                pltpu.sync_copy(
                    msgB.at[pl.ds(0, n)],
                    out_hbm.at[pl.ds(qoff + tid * ROWS_A + off, n)])
                off += n

        @pl.when(tid < NSUB - 1)
        def _():
            _co(ROWS_A)

        @pl.when(tid == NSUB - 1)
        def _():
            _co(N - (NSUB - 1) * ROWS_A)
        plsc.subcore_barrier()

    for p in range(2):
        _zero_msg()
        _zero_acc()
        plsc.subcore_barrier()
        _run_pass(p)


def _sc_aggregate(se, mse, mde, s, stab4, dtab4):
    mesh = plsc.VectorSubcoreMesh(core_axis_name="c", subcore_axis_name="s",
                                  num_cores=NCORE, num_subcores=NSUB)
    kern = pl.kernel(
        _scb_body,
        out_type=jax.ShapeDtypeStruct((4 * N, Q), jnp.float32),
        mesh=mesh,
        compiler_params=pltpu.CompilerParams(use_tc_tiling_on_sc=False,
                                             needs_layout_passes=False),
        scratch_types=[
            pltpu.VMEM((BS * C,), jnp.int32),
            pltpu.VMEM((BS * C,), jnp.int32),
            pltpu.VMEM((BS * C,), jnp.int32),
            pltpu.VMEM((BS * C,), jnp.float32),
            pltpu.VMEM((C, 2 * Q), jnp.float32),
            pltpu.VMEM((C, 3 * Q), jnp.float32),
            pltpu.VMEM((C, Q), jnp.float32),
            pltpu.VMEM((C,), jnp.int32),
            pltpu.VMEM((C,), jnp.int32),
            pltpu.VMEM((C,), jnp.int32),
            pltpu.VMEM_SHARED((ACC_R, Q), jnp.float32),
            pltpu.SemaphoreType.DMA,
            pltpu.SemaphoreType.DMA,
        ],
    )
    return kern(se, mse, mde, s, stab4, dtab4)


def kernel(x, edge_index, W_w, W_b, U_w, U_b, V_w, V_b, D_w, D_b, E_w, E_b):
    d = x.shape[1]

    ut = U_w.T
    wt = W_w.T
    wcat = jnp.concatenate(
        [V_w[:, :d].T, V_w[:, d:].T, D_w.T, E_w.T], axis=1)
    bcat = jnp.concatenate(
        [jnp.zeros((d,), jnp.float32), V_b, D_b, E_b]).reshape(1, 4 * d)
    ub = U_b.reshape(1, d)
    wb = W_b.reshape(1, d)

    stab, dtab = _tc_tables(x, ut, ub, wcat, bcat, wt, wb)
    stab4 = stab.reshape(4 * N, 2 * Q)
    dtab4 = dtab.reshape(4 * N, 3 * Q)

    src_p = edge_index[0].astype(jnp.int32)
    dst_p = edge_index[1].astype(jnp.int32)

    se, mse, mde, s = _sc_scalers(src_p, dst_p)
    out4 = _sc_aggregate(se, mse, mde, s, stab4, dtab4)
    out = out4.reshape(4, N, Q).transpose(1, 0, 2).reshape(N, d)
    return out


# speedup vs baseline: 3.8755x; 1.4346x over previous
import functools

import jax
import jax.numpy as jnp
from jax import lax
from jax.experimental import pallas as pl
from jax.experimental.pallas import tpu as pltpu
from jax.experimental.pallas import tpu_sc as plsc

N = 10000
D = 256
Q = D // 4
SENT = N
NSUB = 16
NCORE = 2
C = 96
HS = 10240
HB = HS // NSUB
ACC_R = N + NSUB
WB = 2000
BS = 8
E_IN = 160000
EPT1 = E_IN // NSUB
ET = 172032
TPT = ET // NSUB
NCH = TPT // C
NBLK = NCH // BS
PAD_OFF = E_IN + N
ROWS_A = 632


def _rsqrt_newton(xx):
    bits = plsc.bitcast(xx, jnp.int32)
    y = plsc.bitcast(
        jnp.int32(0x5F3759DF) - lax.shift_right_logical(bits, 1),
        jnp.float32)
    for _ in range(3):
        y = y * (1.5 - 0.5 * xx * y * y)
    return y


def _tc_body(x_ref, ut_ref, ub_ref, wcat_ref, bcat_ref, wt_ref, wb_ref,
             stab_ref, dtab_ref):
    xb = x_ref[...]
    h = jnp.dot(xb, ut_ref[...], preferred_element_type=jnp.float32)
    h = h + ub_ref[...]
    y = jnp.dot(h, wcat_ref[...], preferred_element_type=jnp.float32)
    y = y + bcat_ref[...]
    xw = jnp.dot(xb, wt_ref[...], preferred_element_type=jnp.float32)
    xw = xw + wb_ref[...]
    a = y[:, 0:D]
    b = y[:, D:2 * D]
    hd = y[:, 2 * D:3 * D]
    he = y[:, 3 * D:4 * D]
    stab_ref[...] = jnp.stack(
        [jnp.concatenate([a[:, q * Q:(q + 1) * Q],
                          hd[:, q * Q:(q + 1) * Q]], axis=1)
         for q in range(4)], axis=0)
    dtab_ref[...] = jnp.stack(
        [jnp.concatenate([b[:, q * Q:(q + 1) * Q],
                          he[:, q * Q:(q + 1) * Q],
                          xw[:, q * Q:(q + 1) * Q]], axis=1)
         for q in range(4)], axis=0)


def _tc_tables(x, ut, ub, wcat, bcat, wt, wb):
    nb = 10
    blk = N // nb
    return pl.pallas_call(
        _tc_body,
        grid=(nb,),
        in_specs=[
            pl.BlockSpec((blk, D), lambda i: (i, 0)),
            pl.BlockSpec((D, D), lambda i: (0, 0)),
            pl.BlockSpec((1, D), lambda i: (0, 0)),
            pl.BlockSpec((D, 4 * D), lambda i: (0, 0)),
            pl.BlockSpec((1, 4 * D), lambda i: (0, 0)),
            pl.BlockSpec((D, D), lambda i: (0, 0)),
            pl.BlockSpec((1, D), lambda i: (0, 0)),
        ],
        out_specs=[
            pl.BlockSpec((4, blk, 2 * Q), lambda i: (0, i, 0)),
            pl.BlockSpec((4, blk, 3 * Q), lambda i: (0, i, 0)),
        ],
        out_shape=[
            jax.ShapeDtypeStruct((4, N, 2 * Q), jnp.float32),
            jax.ShapeDtypeStruct((4, N, 3 * Q), jnp.float32),
        ],
    )(x, ut, ub, wcat, bcat, wt, wb)


def _sca_body(src_hbm, dst_hbm,
              se_hbm, mse_hbm, mde_hbm, s_hbm,
              srcT, dstT, histL, wA, wC, ssum, tmpv,
              staging, histG):
    tid = lax.axis_index("s")
    cid = lax.axis_index("c")
    zero16 = jnp.zeros((16,), jnp.float32)

    base = tid * EPT1
    pltpu.sync_copy(src_hbm.at[pl.ds(base, EPT1)], srcT)
    pltpu.sync_copy(dst_hbm.at[pl.ds(base, EPT1)], dstT)

    def _zh(i, _):
        histL[pl.ds(i * 16, 16)] = zero16
        return 0
    lax.fori_loop(0, HS // 16, _zh, 0)

    def _zs(i, _):
        ssum[pl.ds(i * 16, 16)] = zero16
        return 0
    lax.fori_loop(0, HB // 16, _zs, 0)

    ones16 = jnp.ones((16,), jnp.float32)

    def _blk_a(bi, _):
        def _grp(k, _):
            j = bi * WB + k * 16
            sv = srcT[pl.ds(j, 16)]
            dv = dstT[pl.ds(j, 16)]
            m = sv != dv
            se = jnp.where(m, sv, SENT)
            plsc.addupdate_scatter(histL, [se], ones16)
            wA[pl.ds(k * 16, 16)] = se
            return 0
        lax.fori_loop(0, WB // 16, _grp, 0)

        @pl.when(cid == 0)
        def _():
            pltpu.sync_copy(wA, se_hbm.at[pl.ds(base + bi * WB, WB)])

        def _grp2(k, _):
            j = bi * WB + k * 16
            sv = srcT[pl.ds(j, 16)]
            dv = dstT[pl.ds(j, 16)]
            m = sv != dv
            wA[pl.ds(k * 16, 16)] = jnp.minimum(
                jnp.where(m, sv, SENT), N - 1)
            return 0
        lax.fori_loop(0, WB // 16, _grp2, 0)

        @pl.when(cid == 0)
        def _():
            pltpu.sync_copy(wA, mse_hbm.at[pl.ds(base + bi * WB, WB)])

        def _grp3(k, _):
            j = bi * WB + k * 16
            sv = srcT[pl.ds(j, 16)]
            dv = dstT[pl.ds(j, 16)]
            m = sv != dv
            wA[pl.ds(k * 16, 16)] = jnp.minimum(
                jnp.where(m, dv, SENT), N - 1)
            return 0
        lax.fori_loop(0, WB // 16, _grp3, 0)

        @pl.when(cid == 0)
        def _():
            pltpu.sync_copy(wA, mde_hbm.at[pl.ds(base + bi * WB, WB)])
        return 0
    lax.fori_loop(0, EPT1 // WB, _blk_a, 0)

    pltpu.sync_copy(histL, staging.at[tid])
    plsc.subcore_barrier()

    def _red(j, _):
        pltpu.sync_copy(staging.at[j, pl.ds(tid * HB, HB)], tmpv)

        def _acc(v, _):
            ssum[pl.ds(v * 16, 16)] = (ssum[pl.ds(v * 16, 16)]
                                       + tmpv[pl.ds(v * 16, 16)])
            return 0
        lax.fori_loop(0, HB // 16, _acc, 0)
        return 0
    lax.fori_loop(0, NSUB, _red, 0)
    pltpu.sync_copy(ssum, histG.at[pl.ds(tid * HB, HB)])
    plsc.subcore_barrier()

    pltpu.sync_copy(histG, histL)

    def _rsq(i, _):
        h = histL[pl.ds(i * 16, 16)]
        idx = lax.iota(jnp.int32, 16) + i * 16
        valid = idx < N
        deg = h + jnp.where(valid, 1.0, 0.0)
        y = _rsqrt_newton(jnp.maximum(deg, 1.0))
        histL[pl.ds(i * 16, 16)] = jnp.where(valid, y, 0.0)
        return 0
    lax.fori_loop(0, HS // 16, _rsq, 0)

    def _blk_b(bi, _):
        def _grp(k, _):
            j = bi * WB + k * 16
            sv = srcT[pl.ds(j, 16)]
            dv = dstT[pl.ds(j, 16)]
            m = sv != dv
            se = jnp.where(m, sv, SENT)
            de = jnp.where(m, dv, SENT)
            rs = plsc.load_gather(histL, [se])
            rd = plsc.load_gather(histL, [de])
            wC[pl.ds(k * 16, 16)] = rs * rd
            return 0
        lax.fori_loop(0, WB // 16, _grp, 0)

        @pl.when(cid == 0)
        def _():
            pltpu.sync_copy(wC, s_hbm.at[pl.ds(base + bi * WB, WB)])
        return 0
    lax.fori_loop(0, EPT1 // WB, _blk_b, 0)

    @pl.when(cid == 0)
    def _():
        nrows = jnp.where(tid < NSUB - 1, 0, 0)
        del nrows

        def _self(nrows):
            nch = -(-nrows // 16)

            def _g(k, _):
                node = tid * ROWS_A + k * 16 + lax.iota(jnp.int32, 16)
                node = jnp.minimum(node, N - 1)
                wA[pl.ds(k * 16, 16)] = node
                r = plsc.load_gather(histL, [node])
                wC[pl.ds(k * 16, 16)] = r * r
                return 0
            lax.fori_loop(0, nch, _g, 0)
            off = E_IN + tid * ROWS_A
            pltpu.sync_copy(wA.at[pl.ds(0, nrows)],
                            se_hbm.at[pl.ds(off, nrows)])
            pltpu.sync_copy(wA.at[pl.ds(0, nrows)],
                            mse_hbm.at[pl.ds(off, nrows)])
            pltpu.sync_copy(wA.at[pl.ds(0, nrows)],
                            mde_hbm.at[pl.ds(off, nrows)])
            pltpu.sync_copy(wC.at[pl.ds(0, nrows)],
                            s_hbm.at[pl.ds(off, nrows)])

        @pl.when(tid < NSUB - 1)
        def _():
            _self(ROWS_A)

        @pl.when(tid == NSUB - 1)
        def _():
            _self(N - (NSUB - 1) * ROWS_A)
            npad = ET - PAD_OFF

            def _gp(k, _):
                wA[pl.ds(k * 16, 16)] = jnp.full((16,), SENT, jnp.int32)
                wC[pl.ds(k * 16, 16)] = jnp.zeros((16,), jnp.float32)
                return 0
            lax.fori_loop(0, WB // 16, _gp, 0)
            done = 0
            while done < npad:
                n = min(WB, npad - done)
                pltpu.sync_copy(wA.at[pl.ds(0, n)],
                                se_hbm.at[pl.ds(PAD_OFF + done, n)])
                pltpu.sync_copy(wC.at[pl.ds(0, n)],
                                s_hbm.at[pl.ds(PAD_OFF + done, n)])
                done += n

            def _gq(k, _):
                wA[pl.ds(k * 16, 16)] = jnp.full((16,), N - 1, jnp.int32)
                return 0
            lax.fori_loop(0, WB // 16, _gq, 0)
            done = 0
            while done < npad:
                n = min(WB, npad - done)
                pltpu.sync_copy(wA.at[pl.ds(0, n)],
                                mse_hbm.at[pl.ds(PAD_OFF + done, n)])
                pltpu.sync_copy(wA.at[pl.ds(0, n)],
                                mde_hbm.at[pl.ds(PAD_OFF + done, n)])
                done += n


def _sc_scalers(src_p, dst_p):
    mesh = plsc.VectorSubcoreMesh(core_axis_name="c", subcore_axis_name="s",
                                  num_cores=NCORE, num_subcores=NSUB)
    kern = pl.kernel(
        _sca_body,
        out_type=[
            jax.ShapeDtypeStruct((ET,), jnp.int32),
            jax.ShapeDtypeStruct((ET,), jnp.int32),
            jax.ShapeDtypeStruct((ET,), jnp.int32),
            jax.ShapeDtypeStruct((ET,), jnp.float32),
        ],
        mesh=mesh,
        compiler_params=pltpu.CompilerParams(use_tc_tiling_on_sc=False,
                                             needs_layout_passes=False),
        scratch_types=[
            pltpu.VMEM((EPT1,), jnp.int32),
            pltpu.VMEM((EPT1,), jnp.int32),
            pltpu.VMEM((HS,), jnp.float32),
            pltpu.VMEM((WB,), jnp.int32),
            pltpu.VMEM((WB,), jnp.float32),
            pltpu.VMEM((HB,), jnp.float32),
            pltpu.VMEM((HB,), jnp.float32),
            pltpu.VMEM_SHARED((NSUB, HS), jnp.float32),
            pltpu.VMEM_SHARED((HS,), jnp.float32),
        ],
    )
    return kern(src_p, dst_p)


def _scb_body(se_hbm, mse_hbm, mde_hbm, s_hbm, stab_hbm, dtab_hbm,
              out_hbm,
              seS, mseS, mdeS, sS,
              srows, drows, msgB, idxS, idxD, sidx, seb, sb,
              acc, semGS, semGD, semW):
    tid = lax.axis_index("s")
    cid = lax.axis_index("c")
    zero16 = jnp.zeros((16,), jnp.float32)
    base = tid * TPT

    def _zero_msg():
        def _zm(i, _):
            r = i // (Q // 16)
            c = (i % (Q // 16)) * 16
            msgB[0][r, pl.ds(c, 16)] = zero16
            return 0
        lax.fori_loop(0, C * (Q // 16), _zm, 0)

    def _zero_acc():
        def _za(nrows):
            off = 0
            while off < nrows:
                n = min(C, nrows - off)
                pltpu.sync_copy(msgB[0].at[pl.ds(0, n)],
                                acc.at[pl.ds(tid * ROWS_A + off, n)])
                off += n

        @pl.when(tid < NSUB - 1)
        def _():
            _za(ROWS_A)

        @pl.when(tid == NSUB - 1)
        def _():
            _za(ACC_R - (NSUB - 1) * ROWS_A)

    def _stage(bi, par):
        boff = base + bi * (BS * C)
        pltpu.sync_copy(se_hbm.at[pl.ds(boff, BS * C)], seS[par])
        pltpu.sync_copy(mse_hbm.at[pl.ds(boff, BS * C)], mseS[par])
        pltpu.sync_copy(mde_hbm.at[pl.ds(boff, BS * C)], mdeS[par])
        pltpu.sync_copy(s_hbm.at[pl.ds(boff, BS * C)], sS[par])

    def _run_pass(p):
        qoff = (cid * 2 + p) * N

        def _fill_g(o, par, buf):
            def _f(k, _):
                j = o + k * 16
                idxS[buf][pl.ds(k * 16, 16)] = (
                    qoff + mseS[par][pl.ds(j, 16)])
                idxD[buf][pl.ds(k * 16, 16)] = (
                    qoff + mdeS[par][pl.ds(j, 16)])
                seb[buf][pl.ds(k * 16, 16)] = seS[par][pl.ds(j, 16)]
                sb[buf][pl.ds(k * 16, 16)] = sS[par][pl.ds(j, 16)]
                return 0
            lax.fori_loop(0, C // 16, _f, 0)

        def _fire_g(buf):
            pltpu.async_copy(stab_hbm.at[idxS[buf]], srows[buf],
                             semGS[buf])
            pltpu.async_copy(dtab_hbm.at[idxD[buf]], drows[buf],
                             semGD[buf])

        def _wait_g(buf):
            pltpu.make_async_copy(stab_hbm.at[idxS[buf]], srows[buf],
                                  semGS[buf]).wait()
            pltpu.make_async_copy(dtab_hbm.at[idxD[buf]], drows[buf],
                                  semGD[buf]).wait()

        def _wait_w(buf):
            pltpu.make_async_copy(msgB[buf], acc.at[sidx[buf]],
                                  semW[buf]).wait()

        def _compute(buf):
            def _pe(e, _):
                s = plsc.load_gather(
                    sb[buf], [jnp.full((16,), e, jnp.int32)])
                for v in range(Q // 16):
                    a = srows[buf][e, pl.ds(v * 16, 16)]
                    hd = srows[buf][e, pl.ds(Q + v * 16, 16)]
                    b = drows[buf][e, pl.ds(v * 16, 16)]
                    he = drows[buf][e, pl.ds(Q + v * 16, 16)]
                    xw = drows[buf][e, pl.ds(2 * Q + v * 16, 16)]
                    g = jnp.maximum(a + b, 0.0) + hd + he
                    msgB[buf][e, pl.ds(v * 16, 16)] = xw * (s * g)
                return 0
            lax.fori_loop(0, C, _pe, 0)

        def _fill_sidx(buf):
            def _f(k, _):
                sidx[buf][pl.ds(k * 16, 16)] = seb[buf][pl.ds(k * 16, 16)]
                return 0
            lax.fori_loop(0, C // 16, _f, 0)

        def _fire_w(buf):
            pltpu.async_copy(msgB[buf], acc.at[sidx[buf]], semW[buf],
                             add=True)

        _stage(0, 0)
        _fill_g(0, 0, 0)
        _fire_g(0)
        _fill_g(C, 0, 1)
        _fire_g(1)

        def _block(b, _):
            @pl.when(jnp.logical_and(b + 1 < NBLK, (b + 1) % 2 == 0))
            def _():
                _stage(b + 1, 0)

            @pl.when(jnp.logical_and(b + 1 < NBLK, (b + 1) % 2 == 1))
            def _():
                _stage(b + 1, 1)

            def _pair(i, _):
                for buf in range(2):
                    _wait_g(buf)

                    @pl.when(jnp.logical_or(b > 0, i > 0))
                    def _():
                        _wait_w(buf)
                    _compute(buf)
                    _fill_sidx(buf)
                    _fire_w(buf)
                    last = BS // 2 - 1
                    for par in range(2):
                        @pl.when(jnp.logical_and(i < last, b % 2 == par))
                        def _():
                            _fill_g((2 * i + buf + 2) * C, par, buf)
                            _fire_g(buf)

                        @pl.when(jnp.logical_and(
                            i == last,
                            jnp.logical_and(b + 1 < NBLK,
                                            (b + 1) % 2 == par)))
                        def _():
                            _fill_g(buf * C, par, buf)
                            _fire_g(buf)
                return 0
            lax.fori_loop(0, BS // 2, _pair, 0)
            return 0
        lax.fori_loop(0, NBLK, _block, 0)

        _wait_w(0)
        _wait_w(1)
        plsc.subcore_barrier()

        def _co(nrows):
            off = 0
            while off < nrows:
                n = min(C, nrows - off)
                pltpu.sync_copy(acc.at[pl.ds(tid * ROWS_A + off, n)],
                                msgB[0].at[pl.ds(0, n)])
                pltpu.sync_copy(
                    msgB[0].at[pl.ds(0, n)],
                    out_hbm.at[pl.ds(qoff + tid * ROWS_A + off, n)])
                off += n

        @pl.when(tid < NSUB - 1)
        def _():
            _co(ROWS_A)

        @pl.when(tid == NSUB - 1)
        def _():
            _co(N - (NSUB - 1) * ROWS_A)
        plsc.subcore_barrier()

    for p in range(2):
        _zero_msg()
        _zero_acc()
        plsc.subcore_barrier()
        _run_pass(p)


def _sc_aggregate(se, mse, mde, s, stab4, dtab4):
    mesh = plsc.VectorSubcoreMesh(core_axis_name="c", subcore_axis_name="s",
                                  num_cores=NCORE, num_subcores=NSUB)
    kern = pl.kernel(
        _scb_body,
        out_type=jax.ShapeDtypeStruct((4 * N, Q), jnp.float32),
        mesh=mesh,
        compiler_params=pltpu.CompilerParams(use_tc_tiling_on_sc=False,
                                             needs_layout_passes=False),
        scratch_types=[
            [pltpu.VMEM((BS * C,), jnp.int32)] * 2,
            [pltpu.VMEM((BS * C,), jnp.int32)] * 2,
            [pltpu.VMEM((BS * C,), jnp.int32)] * 2,
            [pltpu.VMEM((BS * C,), jnp.float32)] * 2,
            [pltpu.VMEM((C, 2 * Q), jnp.float32)] * 2,
            [pltpu.VMEM((C, 3 * Q), jnp.float32)] * 2,
            [pltpu.VMEM((C, Q), jnp.float32)] * 2,
            [pltpu.VMEM((C,), jnp.int32)] * 2,
            [pltpu.VMEM((C,), jnp.int32)] * 2,
            [pltpu.VMEM((C,), jnp.int32)] * 2,
            [pltpu.VMEM((C,), jnp.int32)] * 2,
            [pltpu.VMEM((C,), jnp.float32)] * 2,
            pltpu.VMEM_SHARED((ACC_R, Q), jnp.float32),
            [pltpu.SemaphoreType.DMA] * 2,
            [pltpu.SemaphoreType.DMA] * 2,
            [pltpu.SemaphoreType.DMA] * 2,
        ],
    )
    return kern(se, mse, mde, s, stab4, dtab4)


def kernel(x, edge_index, W_w, W_b, U_w, U_b, V_w, V_b, D_w, D_b, E_w, E_b):
    d = x.shape[1]

    ut = U_w.T
    wt = W_w.T
    wcat = jnp.concatenate(
        [V_w[:, :d].T, V_w[:, d:].T, D_w.T, E_w.T], axis=1)
    bcat = jnp.concatenate(
        [jnp.zeros((d,), jnp.float32), V_b, D_b, E_b]).reshape(1, 4 * d)
    ub = U_b.reshape(1, d)
    wb = W_b.reshape(1, d)

    stab, dtab = _tc_tables(x, ut, ub, wcat, bcat, wt, wb)
    stab4 = stab.reshape(4 * N, 2 * Q)
    dtab4 = dtab.reshape(4 * N, 3 * Q)

    src_p = edge_index[0].astype(jnp.int32)
    dst_p = edge_index[1].astype(jnp.int32)

    se, mse, mde, s = _sc_scalers(src_p, dst_p)
    out4 = _sc_aggregate(se, mse, mde, s, stab4, dtab4)
    out = out4.reshape(4, N, Q).transpose(1, 0, 2).reshape(N, d)
    return out


# speedup vs baseline: 5.0977x; 1.3154x over previous
import functools

import jax
import jax.numpy as jnp
from jax import lax
from jax.experimental import pallas as pl
from jax.experimental.pallas import tpu as pltpu
from jax.experimental.pallas import tpu_sc as plsc

N = 10000
D = 256
Q = D // 4
SENT = N
NSUB = 16
NCORE = 2
C = 96
HS = 10240
HB = HS // NSUB
ACC_R = N + NSUB
WB = 2000
BS = 8
E_IN = 160000
EPT1 = E_IN // NSUB
ET = 172032
TPT = ET // NSUB
NCH = TPT // C
NBLK = NCH // BS
PAD_OFF = E_IN + N
ROWS_A = 632


def _rsqrt_newton(xx):
    bits = plsc.bitcast(xx, jnp.int32)
    y = plsc.bitcast(
        jnp.int32(0x5F3759DF) - lax.shift_right_logical(bits, 1),
        jnp.float32)
    for _ in range(3):
        y = y * (1.5 - 0.5 * xx * y * y)
    return y


def _tc_body(x_ref, ut_ref, ub_ref, wcat_ref, bcat_ref, wt_ref, wb_ref,
             stab_ref, dtab_ref):
    xb = x_ref[...]
    h = jnp.dot(xb, ut_ref[...], preferred_element_type=jnp.float32)
    h = h + ub_ref[...]
    y = jnp.dot(h, wcat_ref[...], preferred_element_type=jnp.float32)
    y = y + bcat_ref[...]
    xw = jnp.dot(xb, wt_ref[...], preferred_element_type=jnp.float32)
    xw = xw + wb_ref[...]
    a = y[:, 0:D]
    b = y[:, D:2 * D]
    hd = y[:, 2 * D:3 * D]
    he = y[:, 3 * D:4 * D]
    stab_ref[...] = jnp.stack(
        [jnp.concatenate([a[:, q * Q:(q + 1) * Q],
                          hd[:, q * Q:(q + 1) * Q]], axis=1)
         for q in range(4)], axis=0)
    dtab_ref[...] = jnp.stack(
        [jnp.concatenate([b[:, q * Q:(q + 1) * Q],
                          he[:, q * Q:(q + 1) * Q],
                          xw[:, q * Q:(q + 1) * Q]], axis=1)
         for q in range(4)], axis=0)


def _tc_tables(x, ut, ub, wcat, bcat, wt, wb):
    nb = 10
    blk = N // nb
    return pl.pallas_call(
        _tc_body,
        grid=(nb,),
        in_specs=[
            pl.BlockSpec((blk, D), lambda i: (i, 0)),
            pl.BlockSpec((D, D), lambda i: (0, 0)),
            pl.BlockSpec((1, D), lambda i: (0, 0)),
            pl.BlockSpec((D, 4 * D), lambda i: (0, 0)),
            pl.BlockSpec((1, 4 * D), lambda i: (0, 0)),
            pl.BlockSpec((D, D), lambda i: (0, 0)),
            pl.BlockSpec((1, D), lambda i: (0, 0)),
        ],
        out_specs=[
            pl.BlockSpec((4, blk, 2 * Q), lambda i: (0, i, 0)),
            pl.BlockSpec((4, blk, 3 * Q), lambda i: (0, i, 0)),
        ],
        out_shape=[
            jax.ShapeDtypeStruct((4, N, 2 * Q), jnp.float32),
            jax.ShapeDtypeStruct((4, N, 3 * Q), jnp.float32),
        ],
    )(x, ut, ub, wcat, bcat, wt, wb)


def _sca_body(src_hbm, dst_hbm,
              se_hbm, mse_hbm, mde_hbm, s_hbm,
              srcT, dstT, histL, wA, wC, ssum, tmpv,
              staging, histG):
    tid = lax.axis_index("s")
    cid = lax.axis_index("c")
    zero16 = jnp.zeros((16,), jnp.float32)

    base = tid * EPT1
    pltpu.sync_copy(src_hbm.at[pl.ds(base, EPT1)], srcT)
    pltpu.sync_copy(dst_hbm.at[pl.ds(base, EPT1)], dstT)

    def _zh(i, _):
        histL[pl.ds(i * 16, 16)] = zero16
        return 0
    lax.fori_loop(0, HS // 16, _zh, 0)

    def _zs(i, _):
        ssum[pl.ds(i * 16, 16)] = zero16
        return 0
    lax.fori_loop(0, HB // 16, _zs, 0)

    ones16 = jnp.ones((16,), jnp.float32)

    def _blk_a(bi, _):
        def _grp(k, _):
            j = bi * WB + k * 16
            sv = srcT[pl.ds(j, 16)]
            dv = dstT[pl.ds(j, 16)]
            m = sv != dv
            se = jnp.where(m, sv, SENT)
            plsc.addupdate_scatter(histL, [se], ones16)
            wA[pl.ds(k * 16, 16)] = se
            return 0
        lax.fori_loop(0, WB // 16, _grp, 0)

        @pl.when(cid == 0)
        def _():
            pltpu.sync_copy(wA, se_hbm.at[pl.ds(base + bi * WB, WB)])

        def _grp2(k, _):
            j = bi * WB + k * 16
            sv = srcT[pl.ds(j, 16)]
            dv = dstT[pl.ds(j, 16)]
            m = sv != dv
            wA[pl.ds(k * 16, 16)] = jnp.minimum(
                jnp.where(m, sv, SENT), N - 1)
            return 0
        lax.fori_loop(0, WB // 16, _grp2, 0)

        @pl.when(cid == 0)
        def _():
            pltpu.sync_copy(wA, mse_hbm.at[pl.ds(base + bi * WB, WB)])

        def _grp3(k, _):
            j = bi * WB + k * 16
            sv = srcT[pl.ds(j, 16)]
            dv = dstT[pl.ds(j, 16)]
            m = sv != dv
            wA[pl.ds(k * 16, 16)] = jnp.minimum(
                jnp.where(m, dv, SENT), N - 1)
            return 0
        lax.fori_loop(0, WB // 16, _grp3, 0)

        @pl.when(cid == 0)
        def _():
            pltpu.sync_copy(wA, mde_hbm.at[pl.ds(base + bi * WB, WB)])
        return 0
    lax.fori_loop(0, EPT1 // WB, _blk_a, 0)

    pltpu.sync_copy(histL, staging.at[tid])
    plsc.subcore_barrier()

    def _red(j, _):
        pltpu.sync_copy(staging.at[j, pl.ds(tid * HB, HB)], tmpv)

        def _acc(v, _):
            ssum[pl.ds(v * 16, 16)] = (ssum[pl.ds(v * 16, 16)]
                                       + tmpv[pl.ds(v * 16, 16)])
            return 0
        lax.fori_loop(0, HB // 16, _acc, 0)
        return 0
    lax.fori_loop(0, NSUB, _red, 0)
    pltpu.sync_copy(ssum, histG.at[pl.ds(tid * HB, HB)])
    plsc.subcore_barrier()

    pltpu.sync_copy(histG, histL)

    def _rsq(i, _):
        h = histL[pl.ds(i * 16, 16)]
        idx = lax.iota(jnp.int32, 16) + i * 16
        valid = idx < N
        deg = h + jnp.where(valid, 1.0, 0.0)
        y = _rsqrt_newton(jnp.maximum(deg, 1.0))
        histL[pl.ds(i * 16, 16)] = jnp.where(valid, y, 0.0)
        return 0
    lax.fori_loop(0, HS // 16, _rsq, 0)

    def _blk_b(bi, _):
        def _grp(k, _):
            j = bi * WB + k * 16
            sv = srcT[pl.ds(j, 16)]
            dv = dstT[pl.ds(j, 16)]
            m = sv != dv
            se = jnp.where(m, sv, SENT)
            de = jnp.where(m, dv, SENT)
            rs = plsc.load_gather(histL, [se])
            rd = plsc.load_gather(histL, [de])
            wC[pl.ds(k * 16, 16)] = rs * rd
            return 0
        lax.fori_loop(0, WB // 16, _grp, 0)

        @pl.when(cid == 0)
        def _():
            pltpu.sync_copy(wC, s_hbm.at[pl.ds(base + bi * WB, WB)])
        return 0
    lax.fori_loop(0, EPT1 // WB, _blk_b, 0)

    @pl.when(cid == 0)
    def _():
        nrows = jnp.where(tid < NSUB - 1, 0, 0)
        del nrows

        def _self(nrows):
            nch = -(-nrows // 16)

            def _g(k, _):
                node = tid * ROWS_A + k * 16 + lax.iota(jnp.int32, 16)
                node = jnp.minimum(node, N - 1)
                wA[pl.ds(k * 16, 16)] = node
                r = plsc.load_gather(histL, [node])
                wC[pl.ds(k * 16, 16)] = r * r
                return 0
            lax.fori_loop(0, nch, _g, 0)
            off = E_IN + tid * ROWS_A
            pltpu.sync_copy(wA.at[pl.ds(0, nrows)],
                            se_hbm.at[pl.ds(off, nrows)])
            pltpu.sync_copy(wA.at[pl.ds(0, nrows)],
                            mse_hbm.at[pl.ds(off, nrows)])
            pltpu.sync_copy(wA.at[pl.ds(0, nrows)],
                            mde_hbm.at[pl.ds(off, nrows)])
            pltpu.sync_copy(wC.at[pl.ds(0, nrows)],
                            s_hbm.at[pl.ds(off, nrows)])

        @pl.when(tid < NSUB - 1)
        def _():
            _self(ROWS_A)

        @pl.when(tid == NSUB - 1)
        def _():
            _self(N - (NSUB - 1) * ROWS_A)
            npad = ET - PAD_OFF

            def _gp(k, _):
                wA[pl.ds(k * 16, 16)] = jnp.full((16,), SENT, jnp.int32)
                wC[pl.ds(k * 16, 16)] = jnp.zeros((16,), jnp.float32)
                return 0
            lax.fori_loop(0, WB // 16, _gp, 0)
            done = 0
            while done < npad:
                n = min(WB, npad - done)
                pltpu.sync_copy(wA.at[pl.ds(0, n)],
                                se_hbm.at[pl.ds(PAD_OFF + done, n)])
                pltpu.sync_copy(wC.at[pl.ds(0, n)],
                                s_hbm.at[pl.ds(PAD_OFF + done, n)])
                done += n

            def _gq(k, _):
                wA[pl.ds(k * 16, 16)] = jnp.full((16,), N - 1, jnp.int32)
                return 0
            lax.fori_loop(0, WB // 16, _gq, 0)
            done = 0
            while done < npad:
                n = min(WB, npad - done)
                pltpu.sync_copy(wA.at[pl.ds(0, n)],
                                mse_hbm.at[pl.ds(PAD_OFF + done, n)])
                pltpu.sync_copy(wA.at[pl.ds(0, n)],
                                mde_hbm.at[pl.ds(PAD_OFF + done, n)])
                done += n


def _sc_scalers(src_p, dst_p):
    mesh = plsc.VectorSubcoreMesh(core_axis_name="c", subcore_axis_name="s",
                                  num_cores=NCORE, num_subcores=NSUB)
    kern = pl.kernel(
        _sca_body,
        out_type=[
            jax.ShapeDtypeStruct((ET,), jnp.int32),
            jax.ShapeDtypeStruct((ET,), jnp.int32),
            jax.ShapeDtypeStruct((ET,), jnp.int32),
            jax.ShapeDtypeStruct((ET,), jnp.float32),
        ],
        mesh=mesh,
        compiler_params=pltpu.CompilerParams(use_tc_tiling_on_sc=False,
                                             needs_layout_passes=False),
        scratch_types=[
            pltpu.VMEM((EPT1,), jnp.int32),
            pltpu.VMEM((EPT1,), jnp.int32),
            pltpu.VMEM((HS,), jnp.float32),
            pltpu.VMEM((WB,), jnp.int32),
            pltpu.VMEM((WB,), jnp.float32),
            pltpu.VMEM((HB,), jnp.float32),
            pltpu.VMEM((HB,), jnp.float32),
            pltpu.VMEM_SHARED((NSUB, HS), jnp.float32),
            pltpu.VMEM_SHARED((HS,), jnp.float32),
        ],
    )
    return kern(src_p, dst_p)


def _scb_body(se_hbm, mse_hbm, mde_hbm, s_hbm, stab_hbm, dtab_hbm,
              out_hbm,
              seS, mseS, mdeS, sS,
              srows, drows, msgB, idxS, idxD, sidx, seb, sb,
              acc, semGS, semGD, semW):
    tid = lax.axis_index("s")
    cid = lax.axis_index("c")
    zero16 = jnp.zeros((16,), jnp.float32)
    base = tid * TPT

    def _zero_msg():
        def _zm(i, _):
            r = i // (Q // 16)
            c = (i % (Q // 16)) * 16
            msgB[0][r, pl.ds(c, 16)] = zero16
            return 0
        lax.fori_loop(0, C * (Q // 16), _zm, 0)

    def _zero_acc():
        def _za(nrows):
            off = 0
            while off < nrows:
                n = min(C, nrows - off)
                pltpu.sync_copy(msgB[0].at[pl.ds(0, n)],
                                acc.at[pl.ds(tid * ROWS_A + off, n)])
                off += n

        @pl.when(tid < NSUB - 1)
        def _():
            _za(ROWS_A)

        @pl.when(tid == NSUB - 1)
        def _():
            _za(ACC_R - (NSUB - 1) * ROWS_A)

    def _stage(bi, par):
        boff = base + bi * (BS * C)
        pltpu.sync_copy(se_hbm.at[pl.ds(boff, BS * C)], seS[par])
        pltpu.sync_copy(mse_hbm.at[pl.ds(boff, BS * C)], mseS[par])
        pltpu.sync_copy(mde_hbm.at[pl.ds(boff, BS * C)], mdeS[par])
        pltpu.sync_copy(s_hbm.at[pl.ds(boff, BS * C)], sS[par])

    def _run_pass(p):
        qoff = (cid * 2 + p) * N

        def _fill_g(o, par, buf):
            def _f(k, _):
                j = o + k * 16
                idxS[buf][pl.ds(k * 16, 16)] = (
                    qoff + mseS[par][pl.ds(j, 16)])
                idxD[buf][pl.ds(k * 16, 16)] = (
                    qoff + mdeS[par][pl.ds(j, 16)])
                seb[buf][pl.ds(k * 16, 16)] = seS[par][pl.ds(j, 16)]
                sb[buf][pl.ds(k * 16, 16)] = sS[par][pl.ds(j, 16)]
                return 0
            lax.fori_loop(0, C // 16, _f, 0)

        def _fire_g(buf):
            pltpu.async_copy(stab_hbm.at[idxS[buf]], srows[buf],
                             semGS[buf])
            pltpu.async_copy(dtab_hbm.at[idxD[buf]], drows[buf],
                             semGD[buf])

        def _wait_g(buf):
            pltpu.make_async_copy(stab_hbm.at[idxS[buf]], srows[buf],
                                  semGS[buf]).wait()
            pltpu.make_async_copy(dtab_hbm.at[idxD[buf]], drows[buf],
                                  semGD[buf]).wait()

        def _wait_w(buf):
            pltpu.make_async_copy(msgB[buf], acc.at[sidx[buf]],
                                  semW[buf]).wait()

        def _compute(buf):
            ilv = plsc.PackFormat.INTERLEAVED

            def _pe(e, _):
                s = plsc.load_gather(
                    sb[buf], [jnp.full((16,), e, jnp.int32)])
                for v in range(Q // 32):
                    a2 = srows[buf][e, pl.ds(v * 32, 32)]
                    hd2 = srows[buf][e, pl.ds(Q + v * 32, 32)]
                    b2 = drows[buf][e, pl.ds(v * 32, 32)]
                    he2 = drows[buf][e, pl.ds(Q + v * 32, 32)]
                    xw2 = drows[buf][e, pl.ds(2 * Q + v * 32, 32)]
                    av = plsc.unpack(a2, format=ilv)
                    hdv = plsc.unpack(hd2, format=ilv)
                    bv = plsc.unpack(b2, format=ilv)
                    hev = plsc.unpack(he2, format=ilv)
                    xwv = plsc.unpack(xw2, format=ilv)
                    for h in range(2):
                        g = (jnp.maximum(av[h] + bv[h], 0.0)
                             + hdv[h] + hev[h])
                        msgB[buf][e, pl.ds((2 * v + h) * 16, 16)] = (
                            xwv[h] * (s * g))
                return 0
            lax.fori_loop(0, C, _pe, 0)

        def _fill_sidx(buf):
            def _f(k, _):
                sidx[buf][pl.ds(k * 16, 16)] = seb[buf][pl.ds(k * 16, 16)]
                return 0
            lax.fori_loop(0, C // 16, _f, 0)

        def _fire_w(buf):
            pltpu.async_copy(msgB[buf], acc.at[sidx[buf]], semW[buf],
                             add=True)

        _stage(0, 0)
        _fill_g(0, 0, 0)
        _fire_g(0)
        _fill_g(C, 0, 1)
        _fire_g(1)

        def _block(b, _):
            @pl.when(jnp.logical_and(b + 1 < NBLK, (b + 1) % 2 == 0))
            def _():
                _stage(b + 1, 0)

            @pl.when(jnp.logical_and(b + 1 < NBLK, (b + 1) % 2 == 1))
            def _():
                _stage(b + 1, 1)

            def _pair(i, _):
                for buf in range(2):
                    _wait_g(buf)

                    @pl.when(jnp.logical_or(b > 0, i > 0))
                    def _():
                        _wait_w(buf)
                    _compute(buf)
                    _fill_sidx(buf)
                    _fire_w(buf)
                    last = BS // 2 - 1
                    for par in range(2):
                        @pl.when(jnp.logical_and(i < last, b % 2 == par))
                        def _():
                            _fill_g((2 * i + buf + 2) * C, par, buf)
                            _fire_g(buf)

                        @pl.when(jnp.logical_and(
                            i == last,
                            jnp.logical_and(b + 1 < NBLK,
                                            (b + 1) % 2 == par)))
                        def _():
                            _fill_g(buf * C, par, buf)
                            _fire_g(buf)
                return 0
            lax.fori_loop(0, BS // 2, _pair, 0)
            return 0
        lax.fori_loop(0, NBLK, _block, 0)

        _wait_w(0)
        _wait_w(1)
        plsc.subcore_barrier()

        def _co(nrows):
            off = 0
            while off < nrows:
                n = min(C, nrows - off)
                pltpu.sync_copy(acc.at[pl.ds(tid * ROWS_A + off, n)],
                                msgB[0].at[pl.ds(0, n)])
                pltpu.sync_copy(
                    msgB[0].at[pl.ds(0, n)],
                    out_hbm.at[pl.ds(qoff + tid * ROWS_A + off, n)])
                off += n

        @pl.when(tid < NSUB - 1)
        def _():
            _co(ROWS_A)

        @pl.when(tid == NSUB - 1)
        def _():
            _co(N - (NSUB - 1) * ROWS_A)
        plsc.subcore_barrier()

    for p in range(2):
        _zero_msg()
        _zero_acc()
        plsc.subcore_barrier()
        _run_pass(p)


def _sc_aggregate(se, mse, mde, s, stab4, dtab4):
    mesh = plsc.VectorSubcoreMesh(core_axis_name="c", subcore_axis_name="s",
                                  num_cores=NCORE, num_subcores=NSUB)
    kern = pl.kernel(
        _scb_body,
        out_type=jax.ShapeDtypeStruct((4 * N, Q), jnp.float32),
        mesh=mesh,
        compiler_params=pltpu.CompilerParams(use_tc_tiling_on_sc=False,
                                             needs_layout_passes=False),
        scratch_types=[
            [pltpu.VMEM((BS * C,), jnp.int32)] * 2,
            [pltpu.VMEM((BS * C,), jnp.int32)] * 2,
            [pltpu.VMEM((BS * C,), jnp.int32)] * 2,
            [pltpu.VMEM((BS * C,), jnp.float32)] * 2,
            [pltpu.VMEM((C, 2 * Q), jnp.bfloat16)] * 2,
            [pltpu.VMEM((C, 3 * Q), jnp.bfloat16)] * 2,
            [pltpu.VMEM((C, Q), jnp.float32)] * 2,
            [pltpu.VMEM((C,), jnp.int32)] * 2,
            [pltpu.VMEM((C,), jnp.int32)] * 2,
            [pltpu.VMEM((C,), jnp.int32)] * 2,
            [pltpu.VMEM((C,), jnp.int32)] * 2,
            [pltpu.VMEM((C,), jnp.float32)] * 2,
            pltpu.VMEM_SHARED((ACC_R, Q), jnp.float32),
            [pltpu.SemaphoreType.DMA] * 2,
            [pltpu.SemaphoreType.DMA] * 2,
            [pltpu.SemaphoreType.DMA] * 2,
        ],
    )
    return kern(se, mse, mde, s, stab4, dtab4)


def kernel(x, edge_index, W_w, W_b, U_w, U_b, V_w, V_b, D_w, D_b, E_w, E_b):
    d = x.shape[1]

    ut = U_w.T
    wt = W_w.T
    wcat = jnp.concatenate(
        [V_w[:, :d].T, V_w[:, d:].T, D_w.T, E_w.T], axis=1)
    bcat = jnp.concatenate(
        [jnp.zeros((d,), jnp.float32), V_b, D_b, E_b]).reshape(1, 4 * d)
    ub = U_b.reshape(1, d)
    wb = W_b.reshape(1, d)

    stab, dtab = _tc_tables(x, ut, ub, wcat, bcat, wt, wb)
    stab4 = stab.reshape(4 * N, 2 * Q).astype(jnp.bfloat16)
    dtab4 = dtab.reshape(4 * N, 3 * Q).astype(jnp.bfloat16)

    src_p = edge_index[0].astype(jnp.int32)
    dst_p = edge_index[1].astype(jnp.int32)

    se, mse, mde, s = _sc_scalers(src_p, dst_p)
    out4 = _sc_aggregate(se, mse, mde, s, stab4, dtab4)
    perm = [0] * Q
    for v in range(Q // 32):
        for h in range(2):
            for k in range(16):
                perm[(2 * v + h) * 16 + k] = 32 * v + 2 * k + h
    inv = [0] * Q
    for j, c in enumerate(perm):
        inv[c] = j
    out4 = out4[:, jnp.array(inv, dtype=jnp.int32)]
    out = out4.reshape(4, N, Q).transpose(1, 0, 2).reshape(N, d)
    return out


# speedup vs baseline: 5.4142x; 1.0621x over previous
import functools

import jax
import jax.numpy as jnp
from jax import lax
from jax.experimental import pallas as pl
from jax.experimental.pallas import tpu as pltpu
from jax.experimental.pallas import tpu_sc as plsc

N = 10000
D = 256
Q = D // 4
SENT = N
NSUB = 16
NCORE = 2
C = 128
HS = 10240
HB = HS // NSUB
ACC_R = N + NSUB
WB = 2000
BS = 6
E_IN = 160000
EPT1 = E_IN // NSUB
ET = 172032
TPT = ET // NSUB
NCH = TPT // C
NBLK = NCH // BS
PAD_OFF = E_IN + N
ROWS_A = 632


def _rsqrt_newton(xx):
    bits = plsc.bitcast(xx, jnp.int32)
    y = plsc.bitcast(
        jnp.int32(0x5F3759DF) - lax.shift_right_logical(bits, 1),
        jnp.float32)
    for _ in range(3):
        y = y * (1.5 - 0.5 * xx * y * y)
    return y


def _tc_body(x_ref, ut_ref, ub_ref, wcat_ref, bcat_ref, wt_ref, wb_ref,
             stab_ref, dtab_ref):
    xb = x_ref[...]
    h = jnp.dot(xb, ut_ref[...], preferred_element_type=jnp.float32)
    h = h + ub_ref[...]
    y = jnp.dot(h, wcat_ref[...], preferred_element_type=jnp.float32)
    y = y + bcat_ref[...]
    xw = jnp.dot(xb, wt_ref[...], preferred_element_type=jnp.float32)
    xw = xw + wb_ref[...]
    a = y[:, 0:D]
    b = y[:, D:2 * D]
    hd = y[:, 2 * D:3 * D]
    he = y[:, 3 * D:4 * D]
    stab_ref[...] = jnp.stack(
        [jnp.concatenate([a[:, q * Q:(q + 1) * Q],
                          hd[:, q * Q:(q + 1) * Q]], axis=1)
         for q in range(4)], axis=0)
    dtab_ref[...] = jnp.stack(
        [jnp.concatenate([b[:, q * Q:(q + 1) * Q],
                          he[:, q * Q:(q + 1) * Q],
                          xw[:, q * Q:(q + 1) * Q]], axis=1)
         for q in range(4)], axis=0)


def _tc_tables(x, ut, ub, wcat, bcat, wt, wb):
    nb = 10
    blk = N // nb
    return pl.pallas_call(
        _tc_body,
        grid=(nb,),
        in_specs=[
            pl.BlockSpec((blk, D), lambda i: (i, 0)),
            pl.BlockSpec((D, D), lambda i: (0, 0)),
            pl.BlockSpec((1, D), lambda i: (0, 0)),
            pl.BlockSpec((D, 4 * D), lambda i: (0, 0)),
            pl.BlockSpec((1, 4 * D), lambda i: (0, 0)),
            pl.BlockSpec((D, D), lambda i: (0, 0)),
            pl.BlockSpec((1, D), lambda i: (0, 0)),
        ],
        out_specs=[
            pl.BlockSpec((4, blk, 2 * Q), lambda i: (0, i, 0)),
            pl.BlockSpec((4, blk, 3 * Q), lambda i: (0, i, 0)),
        ],
        out_shape=[
            jax.ShapeDtypeStruct((4, N, 2 * Q), jnp.float32),
            jax.ShapeDtypeStruct((4, N, 3 * Q), jnp.float32),
        ],
    )(x, ut, ub, wcat, bcat, wt, wb)


def _sca_body(src_hbm, dst_hbm,
              se_hbm, mse_hbm, mde_hbm, s_hbm,
              srcT, dstT, histL, wA, wC, ssum, tmpv,
              staging, histG):
    tid = lax.axis_index("s")
    cid = lax.axis_index("c")
    zero16 = jnp.zeros((16,), jnp.float32)

    base = tid * EPT1
    pltpu.sync_copy(src_hbm.at[pl.ds(base, EPT1)], srcT)
    pltpu.sync_copy(dst_hbm.at[pl.ds(base, EPT1)], dstT)

    def _zh(i, _):
        histL[pl.ds(i * 16, 16)] = zero16
        return 0
    lax.fori_loop(0, HS // 16, _zh, 0)

    def _zs(i, _):
        ssum[pl.ds(i * 16, 16)] = zero16
        return 0
    lax.fori_loop(0, HB // 16, _zs, 0)

    ones16 = jnp.ones((16,), jnp.float32)

    def _blk_a(bi, _):
        def _grp(k, _):
            j = bi * WB + k * 16
            sv = srcT[pl.ds(j, 16)]
            dv = dstT[pl.ds(j, 16)]
            m = sv != dv
            se = jnp.where(m, sv, SENT)
            plsc.addupdate_scatter(histL, [se], ones16)
            wA[pl.ds(k * 16, 16)] = se
            return 0
        lax.fori_loop(0, WB // 16, _grp, 0)

        @pl.when(cid == 0)
        def _():
            pltpu.sync_copy(wA, se_hbm.at[pl.ds(base + bi * WB, WB)])

        def _grp2(k, _):
            j = bi * WB + k * 16
            sv = srcT[pl.ds(j, 16)]
            dv = dstT[pl.ds(j, 16)]
            m = sv != dv
            wA[pl.ds(k * 16, 16)] = jnp.minimum(
                jnp.where(m, sv, SENT), N - 1)
            return 0
        lax.fori_loop(0, WB // 16, _grp2, 0)

        @pl.when(cid == 0)
        def _():
            pltpu.sync_copy(wA, mse_hbm.at[pl.ds(base + bi * WB, WB)])

        def _grp3(k, _):
            j = bi * WB + k * 16
            sv = srcT[pl.ds(j, 16)]
            dv = dstT[pl.ds(j, 16)]
            m = sv != dv
            wA[pl.ds(k * 16, 16)] = jnp.minimum(
                jnp.where(m, dv, SENT), N - 1)
            return 0
        lax.fori_loop(0, WB // 16, _grp3, 0)

        @pl.when(cid == 0)
        def _():
            pltpu.sync_copy(wA, mde_hbm.at[pl.ds(base + bi * WB, WB)])
        return 0
    lax.fori_loop(0, EPT1 // WB, _blk_a, 0)

    pltpu.sync_copy(histL, staging.at[tid])
    plsc.subcore_barrier()

    def _red(j, _):
        pltpu.sync_copy(staging.at[j, pl.ds(tid * HB, HB)], tmpv)

        def _acc(v, _):
            ssum[pl.ds(v * 16, 16)] = (ssum[pl.ds(v * 16, 16)]
                                       + tmpv[pl.ds(v * 16, 16)])
            return 0
        lax.fori_loop(0, HB // 16, _acc, 0)
        return 0
    lax.fori_loop(0, NSUB, _red, 0)
    pltpu.sync_copy(ssum, histG.at[pl.ds(tid * HB, HB)])
    plsc.subcore_barrier()

    pltpu.sync_copy(histG, histL)

    def _rsq(i, _):
        h = histL[pl.ds(i * 16, 16)]
        idx = lax.iota(jnp.int32, 16) + i * 16
        valid = idx < N
        deg = h + jnp.where(valid, 1.0, 0.0)
        y = _rsqrt_newton(jnp.maximum(deg, 1.0))
        histL[pl.ds(i * 16, 16)] = jnp.where(valid, y, 0.0)
        return 0
    lax.fori_loop(0, HS // 16, _rsq, 0)

    def _blk_b(bi, _):
        def _grp(k, _):
            j = bi * WB + k * 16
            sv = srcT[pl.ds(j, 16)]
            dv = dstT[pl.ds(j, 16)]
            m = sv != dv
            se = jnp.where(m, sv, SENT)
            de = jnp.where(m, dv, SENT)
            rs = plsc.load_gather(histL, [se])
            rd = plsc.load_gather(histL, [de])
            wC[pl.ds(k * 16, 16)] = rs * rd
            return 0
        lax.fori_loop(0, WB // 16, _grp, 0)

        @pl.when(cid == 0)
        def _():
            pltpu.sync_copy(wC, s_hbm.at[pl.ds(base + bi * WB, WB)])
        return 0
    lax.fori_loop(0, EPT1 // WB, _blk_b, 0)

    @pl.when(cid == 0)
    def _():
        nrows = jnp.where(tid < NSUB - 1, 0, 0)
        del nrows

        def _self(nrows):
            nch = -(-nrows // 16)

            def _g(k, _):
                node = tid * ROWS_A + k * 16 + lax.iota(jnp.int32, 16)
                node = jnp.minimum(node, N - 1)
                wA[pl.ds(k * 16, 16)] = node
                r = plsc.load_gather(histL, [node])
                wC[pl.ds(k * 16, 16)] = r * r
                return 0
            lax.fori_loop(0, nch, _g, 0)
            off = E_IN + tid * ROWS_A
            pltpu.sync_copy(wA.at[pl.ds(0, nrows)],
                            se_hbm.at[pl.ds(off, nrows)])
            pltpu.sync_copy(wA.at[pl.ds(0, nrows)],
                            mse_hbm.at[pl.ds(off, nrows)])
            pltpu.sync_copy(wA.at[pl.ds(0, nrows)],
                            mde_hbm.at[pl.ds(off, nrows)])
            pltpu.sync_copy(wC.at[pl.ds(0, nrows)],
                            s_hbm.at[pl.ds(off, nrows)])

        @pl.when(tid < NSUB - 1)
        def _():
            _self(ROWS_A)

        @pl.when(tid == NSUB - 1)
        def _():
            _self(N - (NSUB - 1) * ROWS_A)
            npad = ET - PAD_OFF

            def _gp(k, _):
                wA[pl.ds(k * 16, 16)] = jnp.full((16,), SENT, jnp.int32)
                wC[pl.ds(k * 16, 16)] = jnp.zeros((16,), jnp.float32)
                return 0
            lax.fori_loop(0, WB // 16, _gp, 0)
            done = 0
            while done < npad:
                n = min(WB, npad - done)
                pltpu.sync_copy(wA.at[pl.ds(0, n)],
                                se_hbm.at[pl.ds(PAD_OFF + done, n)])
                pltpu.sync_copy(wC.at[pl.ds(0, n)],
                                s_hbm.at[pl.ds(PAD_OFF + done, n)])
                done += n

            def _gq(k, _):
                wA[pl.ds(k * 16, 16)] = jnp.full((16,), N - 1, jnp.int32)
                return 0
            lax.fori_loop(0, WB // 16, _gq, 0)
            done = 0
            while done < npad:
                n = min(WB, npad - done)
                pltpu.sync_copy(wA.at[pl.ds(0, n)],
                                mse_hbm.at[pl.ds(PAD_OFF + done, n)])
                pltpu.sync_copy(wA.at[pl.ds(0, n)],
                                mde_hbm.at[pl.ds(PAD_OFF + done, n)])
                done += n


def _sc_scalers(src_p, dst_p):
    mesh = plsc.VectorSubcoreMesh(core_axis_name="c", subcore_axis_name="s",
                                  num_cores=NCORE, num_subcores=NSUB)
    kern = pl.kernel(
        _sca_body,
        out_type=[
            jax.ShapeDtypeStruct((ET,), jnp.int32),
            jax.ShapeDtypeStruct((ET,), jnp.int32),
            jax.ShapeDtypeStruct((ET,), jnp.int32),
            jax.ShapeDtypeStruct((ET,), jnp.float32),
        ],
        mesh=mesh,
        compiler_params=pltpu.CompilerParams(use_tc_tiling_on_sc=False,
                                             needs_layout_passes=False),
        scratch_types=[
            pltpu.VMEM((EPT1,), jnp.int32),
            pltpu.VMEM((EPT1,), jnp.int32),
            pltpu.VMEM((HS,), jnp.float32),
            pltpu.VMEM((WB,), jnp.int32),
            pltpu.VMEM((WB,), jnp.float32),
            pltpu.VMEM((HB,), jnp.float32),
            pltpu.VMEM((HB,), jnp.float32),
            pltpu.VMEM_SHARED((NSUB, HS), jnp.float32),
            pltpu.VMEM_SHARED((HS,), jnp.float32),
        ],
    )
    return kern(src_p, dst_p)


def _scb_body(se_hbm, mse_hbm, mde_hbm, s_hbm, stab_hbm, dtab_hbm,
              out_hbm,
              seS, mseS, mdeS, sS,
              srows, drows, msgB, idxS, idxD, sidx, seb, sb,
              acc, semGS, semGD, semW, semT):
    tid = lax.axis_index("s")
    cid = lax.axis_index("c")
    zero16 = jnp.zeros((16,), jnp.float32)
    base = tid * TPT

    def _zero_msg():
        def _zm(i, _):
            r = i // (Q // 16)
            c = (i % (Q // 16)) * 16
            msgB[0][r, pl.ds(c, 16)] = zero16
            return 0
        lax.fori_loop(0, C * (Q // 16), _zm, 0)

    def _zero_acc():
        def _za(nrows):
            off = 0
            while off < nrows:
                n = min(C, nrows - off)
                pltpu.sync_copy(msgB[0].at[pl.ds(0, n)],
                                acc.at[pl.ds(tid * ROWS_A + off, n)])
                off += n

        @pl.when(tid < NSUB - 1)
        def _():
            _za(ROWS_A)

        @pl.when(tid == NSUB - 1)
        def _():
            _za(ACC_R - (NSUB - 1) * ROWS_A)

    def _stage_fire(bi, par):
        boff = bi * (BS * C)
        pltpu.async_copy(se_hbm.at[pl.ds(base + boff, BS * C)], seS[par],
                         semT[par])
        pltpu.async_copy(mse_hbm.at[pl.ds(base + boff, BS * C)],
                         mseS[par], semT[par])
        pltpu.async_copy(mde_hbm.at[pl.ds(base + boff, BS * C)],
                         mdeS[par], semT[par])
        pltpu.async_copy(s_hbm.at[pl.ds(base + boff, BS * C)], sS[par],
                         semT[par])

    def _stage_wait(bi, par):
        boff = bi * (BS * C)
        pltpu.make_async_copy(se_hbm.at[pl.ds(base + boff, BS * C)],
                              seS[par], semT[par]).wait()
        pltpu.make_async_copy(mse_hbm.at[pl.ds(base + boff, BS * C)],
                              mseS[par], semT[par]).wait()
        pltpu.make_async_copy(mde_hbm.at[pl.ds(base + boff, BS * C)],
                              mdeS[par], semT[par]).wait()
        pltpu.make_async_copy(s_hbm.at[pl.ds(base + boff, BS * C)],
                              sS[par], semT[par]).wait()

    def _run_pass(p):
        qoff = (cid * 2 + p) * N

        def _fill_g(o, par, buf):
            def _f(k, _):
                j = o + k * 16
                idxS[buf][pl.ds(k * 16, 16)] = (
                    qoff + mseS[par][pl.ds(j, 16)])
                idxD[buf][pl.ds(k * 16, 16)] = (
                    qoff + mdeS[par][pl.ds(j, 16)])
                seb[buf][pl.ds(k * 16, 16)] = seS[par][pl.ds(j, 16)]
                sb[buf][pl.ds(k * 16, 16)] = sS[par][pl.ds(j, 16)]
                return 0
            lax.fori_loop(0, C // 16, _f, 0)

        def _fire_g(buf):
            pltpu.async_copy(stab_hbm.at[idxS[buf]], srows[buf],
                             semGS[buf])
            pltpu.async_copy(dtab_hbm.at[idxD[buf]], drows[buf],
                             semGD[buf])

        def _wait_g(buf):
            pltpu.make_async_copy(stab_hbm.at[idxS[buf]], srows[buf],
                                  semGS[buf]).wait()
            pltpu.make_async_copy(dtab_hbm.at[idxD[buf]], drows[buf],
                                  semGD[buf]).wait()

        def _wait_w(buf):
            pltpu.make_async_copy(msgB[buf], acc.at[sidx[buf]],
                                  semW[buf]).wait()

        def _compute(buf):
            ilv = plsc.PackFormat.INTERLEAVED

            def _one(e):
                s = plsc.load_gather(
                    sb[buf], [jnp.full((16,), e, jnp.int32)])
                for v in range(Q // 32):
                    a2 = srows[buf][e, pl.ds(v * 32, 32)]
                    hd2 = srows[buf][e, pl.ds(Q + v * 32, 32)]
                    b2 = drows[buf][e, pl.ds(v * 32, 32)]
                    he2 = drows[buf][e, pl.ds(Q + v * 32, 32)]
                    xw2 = drows[buf][e, pl.ds(2 * Q + v * 32, 32)]
                    av = plsc.unpack(a2, format=ilv)
                    hdv = plsc.unpack(hd2, format=ilv)
                    bv = plsc.unpack(b2, format=ilv)
                    hev = plsc.unpack(he2, format=ilv)
                    xwv = plsc.unpack(xw2, format=ilv)
                    for h in range(2):
                        g = (jnp.maximum(av[h] + bv[h], 0.0)
                             + hdv[h] + hev[h])
                        msgB[buf][e, pl.ds((2 * v + h) * 16, 16)] = (
                            xwv[h] * (s * g))

            def _pe(j, _):
                _one(2 * j)
                _one(2 * j + 1)
                return 0
            lax.fori_loop(0, C // 2, _pe, 0)

        def _fill_sidx(buf):
            def _f(k, _):
                sidx[buf][pl.ds(k * 16, 16)] = seb[buf][pl.ds(k * 16, 16)]
                return 0
            lax.fori_loop(0, C // 16, _f, 0)

        def _fire_w(buf):
            pltpu.async_copy(msgB[buf], acc.at[sidx[buf]], semW[buf],
                             add=True)

        _stage_fire(0, 0)
        _stage_wait(0, 0)
        _stage_fire(1, 1)
        _fill_g(0, 0, 0)
        _fire_g(0)
        _fill_g(C, 0, 1)
        _fire_g(1)

        def _block(b, _):
            for par in range(2):
                @pl.when(jnp.logical_and(b > 0, b % 2 == par))
                def _():
                    _stage_wait(b, par)

            def _pair(i, _):
                for buf in range(2):
                    _wait_g(buf)

                    @pl.when(jnp.logical_or(b > 0, i > 0))
                    def _():
                        _wait_w(buf)
                    _compute(buf)
                    _fill_sidx(buf)
                    _fire_w(buf)
                    last = BS // 2 - 1
                    for par in range(2):
                        @pl.when(jnp.logical_and(i < last, b % 2 == par))
                        def _():
                            _fill_g((2 * i + buf + 2) * C, par, buf)
                            _fire_g(buf)

                        @pl.when(jnp.logical_and(
                            i == last,
                            jnp.logical_and(b + 1 < NBLK,
                                            (b + 1) % 2 == par)))
                        def _():
                            _fill_g(buf * C, par, buf)
                            _fire_g(buf)
                return 0
            lax.fori_loop(0, BS // 2, _pair, 0)

            for par in range(2):
                @pl.when(jnp.logical_and(b + 2 < NBLK, b % 2 == par))
                def _():
                    _stage_fire(b + 2, par)
            return 0
        lax.fori_loop(0, NBLK, _block, 0)

        _wait_w(0)
        _wait_w(1)
        plsc.subcore_barrier()

        def _co(nrows):
            off = 0
            while off < nrows:
                n = min(C, nrows - off)
                pltpu.sync_copy(acc.at[pl.ds(tid * ROWS_A + off, n)],
                                msgB[0].at[pl.ds(0, n)])
                pltpu.sync_copy(
                    msgB[0].at[pl.ds(0, n)],
                    out_hbm.at[pl.ds(qoff + tid * ROWS_A + off, n)])
                off += n

        @pl.when(tid < NSUB - 1)
        def _():
            _co(ROWS_A)

        @pl.when(tid == NSUB - 1)
        def _():
            _co(N - (NSUB - 1) * ROWS_A)
        plsc.subcore_barrier()

    for p in range(2):
        _zero_msg()
        _zero_acc()
        plsc.subcore_barrier()
        _run_pass(p)


def _sc_aggregate(se, mse, mde, s, stab4, dtab4):
    mesh = plsc.VectorSubcoreMesh(core_axis_name="c", subcore_axis_name="s",
                                  num_cores=NCORE, num_subcores=NSUB)
    kern = pl.kernel(
        _scb_body,
        out_type=jax.ShapeDtypeStruct((4 * N, Q), jnp.float32),
        mesh=mesh,
        compiler_params=pltpu.CompilerParams(use_tc_tiling_on_sc=False,
                                             needs_layout_passes=False),
        scratch_types=[
            [pltpu.VMEM((BS * C,), jnp.int32)] * 2,
            [pltpu.VMEM((BS * C,), jnp.int32)] * 2,
            [pltpu.VMEM((BS * C,), jnp.int32)] * 2,
            [pltpu.VMEM((BS * C,), jnp.float32)] * 2,
            [pltpu.VMEM((C, 2 * Q), jnp.bfloat16)] * 2,
            [pltpu.VMEM((C, 3 * Q), jnp.bfloat16)] * 2,
            [pltpu.VMEM((C, Q), jnp.float32)] * 2,
            [pltpu.VMEM((C,), jnp.int32)] * 2,
            [pltpu.VMEM((C,), jnp.int32)] * 2,
            [pltpu.VMEM((C,), jnp.int32)] * 2,
            [pltpu.VMEM((C,), jnp.int32)] * 2,
            [pltpu.VMEM((C,), jnp.float32)] * 2,
            pltpu.VMEM_SHARED((ACC_R, Q), jnp.float32),
            [pltpu.SemaphoreType.DMA] * 2,
            [pltpu.SemaphoreType.DMA] * 2,
            [pltpu.SemaphoreType.DMA] * 2,
            [pltpu.SemaphoreType.DMA] * 2,
        ],
    )
    return kern(se, mse, mde, s, stab4, dtab4)


def kernel(x, edge_index, W_w, W_b, U_w, U_b, V_w, V_b, D_w, D_b, E_w, E_b):
    d = x.shape[1]

    ut = U_w.T
    wt = W_w.T
    wcat = jnp.concatenate(
        [V_w[:, :d].T, V_w[:, d:].T, D_w.T, E_w.T], axis=1)
    bcat = jnp.concatenate(
        [jnp.zeros((d,), jnp.float32), V_b, D_b, E_b]).reshape(1, 4 * d)
    ub = U_b.reshape(1, d)
    wb = W_b.reshape(1, d)

    stab, dtab = _tc_tables(x, ut, ub, wcat, bcat, wt, wb)
    stab4 = stab.reshape(4 * N, 2 * Q).astype(jnp.bfloat16)
    dtab4 = dtab.reshape(4 * N, 3 * Q).astype(jnp.bfloat16)

    src_p = edge_index[0].astype(jnp.int32)
    dst_p = edge_index[1].astype(jnp.int32)

    se, mse, mde, s = _sc_scalers(src_p, dst_p)
    out4 = _sc_aggregate(se, mse, mde, s, stab4, dtab4)
    perm = [0] * Q
    for v in range(Q // 32):
        for h in range(2):
            for k in range(16):
                perm[(2 * v + h) * 16 + k] = 32 * v + 2 * k + h
    inv = [0] * Q
    for j, c in enumerate(perm):
        inv[c] = j
    out4 = out4[:, jnp.array(inv, dtype=jnp.int32)]
    out = out4.reshape(4, N, Q).transpose(1, 0, 2).reshape(N, d)
    return out


# speedup vs baseline: 5.4214x; 1.0013x over previous
import functools

import jax
import jax.numpy as jnp
from jax import lax
from jax.experimental import pallas as pl
from jax.experimental.pallas import tpu as pltpu
from jax.experimental.pallas import tpu_sc as plsc

N = 10000
D = 256
Q = D // 4
SENT = N
NSUB = 16
NCORE = 2
C = 128
HS = 10240
HB = HS // NSUB
ACC_R = N + NSUB
WB = 2000
BS = 6
E_IN = 160000
EPT1 = E_IN // NSUB
ET = 172032
TPT = ET // NSUB
NCH = TPT // C
NBLK = NCH // BS
PAD_OFF = E_IN + N
ROWS_A = 632


def _rsqrt_newton(xx):
    bits = plsc.bitcast(xx, jnp.int32)
    y = plsc.bitcast(
        jnp.int32(0x5F3759DF) - lax.shift_right_logical(bits, 1),
        jnp.float32)
    for _ in range(3):
        y = y * (1.5 - 0.5 * xx * y * y)
    return y


def _tc_body(x_ref, ut_ref, ub_ref, wcat_ref, bcat_ref, wt_ref, wb_ref,
             stab_ref, dtab_ref):
    xb = x_ref[...]
    h = jnp.dot(xb, ut_ref[...], preferred_element_type=jnp.float32)
    h = h + ub_ref[...]
    y = jnp.dot(h, wcat_ref[...], preferred_element_type=jnp.float32)
    y = y + bcat_ref[...]
    xw = jnp.dot(xb, wt_ref[...], preferred_element_type=jnp.float32)
    xw = xw + wb_ref[...]
    a = y[:, 0:D]
    b = y[:, D:2 * D]
    hd = y[:, 2 * D:3 * D]
    he = y[:, 3 * D:4 * D]
    stab_ref[...] = jnp.stack(
        [jnp.concatenate([a[:, q * Q:(q + 1) * Q],
                          hd[:, q * Q:(q + 1) * Q]], axis=1)
         for q in range(4)], axis=0)
    dtab_ref[...] = jnp.stack(
        [jnp.concatenate([b[:, q * Q:(q + 1) * Q],
                          he[:, q * Q:(q + 1) * Q],
                          xw[:, q * Q:(q + 1) * Q]], axis=1)
         for q in range(4)], axis=0)


def _tc_tables(x, ut, ub, wcat, bcat, wt, wb):
    nb = 10
    blk = N // nb
    return pl.pallas_call(
        _tc_body,
        grid=(nb,),
        in_specs=[
            pl.BlockSpec((blk, D), lambda i: (i, 0)),
            pl.BlockSpec((D, D), lambda i: (0, 0)),
            pl.BlockSpec((1, D), lambda i: (0, 0)),
            pl.BlockSpec((D, 4 * D), lambda i: (0, 0)),
            pl.BlockSpec((1, 4 * D), lambda i: (0, 0)),
            pl.BlockSpec((D, D), lambda i: (0, 0)),
            pl.BlockSpec((1, D), lambda i: (0, 0)),
        ],
        out_specs=[
            pl.BlockSpec((4, blk, 2 * Q), lambda i: (0, i, 0)),
            pl.BlockSpec((4, blk, 3 * Q), lambda i: (0, i, 0)),
        ],
        out_shape=[
            jax.ShapeDtypeStruct((4, N, 2 * Q), jnp.float32),
            jax.ShapeDtypeStruct((4, N, 3 * Q), jnp.float32),
        ],
    )(x, ut, ub, wcat, bcat, wt, wb)


def _sca_body(src_hbm, dst_hbm,
              se_hbm, mse_hbm, mde_hbm, s_hbm,
              srcT, dstT, histL, wA, wC, ssum, tmpv,
              staging, histG):
    tid = lax.axis_index("s")
    cid = lax.axis_index("c")
    zero16 = jnp.zeros((16,), jnp.float32)

    base = tid * EPT1
    pltpu.sync_copy(src_hbm.at[pl.ds(base, EPT1)], srcT)
    pltpu.sync_copy(dst_hbm.at[pl.ds(base, EPT1)], dstT)

    def _zh(i, _):
        histL[pl.ds(i * 16, 16)] = zero16
        return 0
    lax.fori_loop(0, HS // 16, _zh, 0)

    def _zs(i, _):
        ssum[pl.ds(i * 16, 16)] = zero16
        return 0
    lax.fori_loop(0, HB // 16, _zs, 0)

    ones16 = jnp.ones((16,), jnp.float32)

    def _blk_a(bi, _):
        def _grp(k, _):
            j = bi * WB + k * 16
            sv = srcT[pl.ds(j, 16)]
            dv = dstT[pl.ds(j, 16)]
            m = sv != dv
            se = jnp.where(m, sv, SENT)
            plsc.addupdate_scatter(histL, [se], ones16)
            wA[pl.ds(k * 16, 16)] = se
            return 0
        lax.fori_loop(0, WB // 16, _grp, 0)

        @pl.when(cid == 0)
        def _():
            pltpu.sync_copy(wA, se_hbm.at[pl.ds(base + bi * WB, WB)])

        def _grp2(k, _):
            j = bi * WB + k * 16
            sv = srcT[pl.ds(j, 16)]
            dv = dstT[pl.ds(j, 16)]
            m = sv != dv
            wA[pl.ds(k * 16, 16)] = jnp.minimum(
                jnp.where(m, sv, SENT), N - 1)
            return 0
        lax.fori_loop(0, WB // 16, _grp2, 0)

        @pl.when(cid == 0)
        def _():
            pltpu.sync_copy(wA, mse_hbm.at[pl.ds(base + bi * WB, WB)])

        def _grp3(k, _):
            j = bi * WB + k * 16
            sv = srcT[pl.ds(j, 16)]
            dv = dstT[pl.ds(j, 16)]
            m = sv != dv
            wA[pl.ds(k * 16, 16)] = jnp.minimum(
                jnp.where(m, dv, SENT), N - 1)
            return 0
        lax.fori_loop(0, WB // 16, _grp3, 0)

        @pl.when(cid == 0)
        def _():
            pltpu.sync_copy(wA, mde_hbm.at[pl.ds(base + bi * WB, WB)])
        return 0
    lax.fori_loop(0, EPT1 // WB, _blk_a, 0)

    pltpu.sync_copy(histL, staging.at[tid])
    plsc.subcore_barrier()

    pltpu.sync_copy(staging.at[:, pl.ds(tid * HB, HB)], tmpv)

    def _red(j, _):
        def _acc(v, _):
            ssum[pl.ds(v * 16, 16)] = (ssum[pl.ds(v * 16, 16)]
                                       + tmpv[j, pl.ds(v * 16, 16)])
            return 0
        lax.fori_loop(0, HB // 16, _acc, 0)
        return 0
    lax.fori_loop(0, NSUB, _red, 0)
    pltpu.sync_copy(ssum, histG.at[pl.ds(tid * HB, HB)])
    plsc.subcore_barrier()

    pltpu.sync_copy(histG, histL)

    def _rsq(i, _):
        h = histL[pl.ds(i * 16, 16)]
        idx = lax.iota(jnp.int32, 16) + i * 16
        valid = idx < N
        deg = h + jnp.where(valid, 1.0, 0.0)
        y = _rsqrt_newton(jnp.maximum(deg, 1.0))
        histL[pl.ds(i * 16, 16)] = jnp.where(valid, y, 0.0)
        return 0
    lax.fori_loop(0, HS // 16, _rsq, 0)

    def _blk_b(bi, _):
        def _grp(k, _):
            j = bi * WB + k * 16
            sv = srcT[pl.ds(j, 16)]
            dv = dstT[pl.ds(j, 16)]
            m = sv != dv
            se = jnp.where(m, sv, SENT)
            de = jnp.where(m, dv, SENT)
            rs = plsc.load_gather(histL, [se])
            rd = plsc.load_gather(histL, [de])
            wC[pl.ds(k * 16, 16)] = rs * rd
            return 0
        lax.fori_loop(0, WB // 16, _grp, 0)

        @pl.when(cid == 0)
        def _():
            pltpu.sync_copy(wC, s_hbm.at[pl.ds(base + bi * WB, WB)])
        return 0
    lax.fori_loop(0, EPT1 // WB, _blk_b, 0)

    @pl.when(cid == 0)
    def _():
        nrows = jnp.where(tid < NSUB - 1, 0, 0)
        del nrows

        def _self(nrows):
            nch = -(-nrows // 16)

            def _g(k, _):
                node = tid * ROWS_A + k * 16 + lax.iota(jnp.int32, 16)
                node = jnp.minimum(node, N - 1)
                wA[pl.ds(k * 16, 16)] = node
                r = plsc.load_gather(histL, [node])
                wC[pl.ds(k * 16, 16)] = r * r
                return 0
            lax.fori_loop(0, nch, _g, 0)
            off = E_IN + tid * ROWS_A
            pltpu.sync_copy(wA.at[pl.ds(0, nrows)],
                            se_hbm.at[pl.ds(off, nrows)])
            pltpu.sync_copy(wA.at[pl.ds(0, nrows)],
                            mse_hbm.at[pl.ds(off, nrows)])
            pltpu.sync_copy(wA.at[pl.ds(0, nrows)],
                            mde_hbm.at[pl.ds(off, nrows)])
            pltpu.sync_copy(wC.at[pl.ds(0, nrows)],
                            s_hbm.at[pl.ds(off, nrows)])

        @pl.when(tid < NSUB - 1)
        def _():
            _self(ROWS_A)

        @pl.when(tid == NSUB - 1)
        def _():
            _self(N - (NSUB - 1) * ROWS_A)
            npad = ET - PAD_OFF

            def _gp(k, _):
                wA[pl.ds(k * 16, 16)] = jnp.full((16,), SENT, jnp.int32)
                wC[pl.ds(k * 16, 16)] = jnp.zeros((16,), jnp.float32)
                return 0
            lax.fori_loop(0, WB // 16, _gp, 0)
            done = 0
            while done < npad:
                n = min(WB, npad - done)
                pltpu.sync_copy(wA.at[pl.ds(0, n)],
                                se_hbm.at[pl.ds(PAD_OFF + done, n)])
                pltpu.sync_copy(wC.at[pl.ds(0, n)],
                                s_hbm.at[pl.ds(PAD_OFF + done, n)])
                done += n

            def _gq(k, _):
                wA[pl.ds(k * 16, 16)] = jnp.full((16,), N - 1, jnp.int32)
                return 0
            lax.fori_loop(0, WB // 16, _gq, 0)
            done = 0
            while done < npad:
                n = min(WB, npad - done)
                pltpu.sync_copy(wA.at[pl.ds(0, n)],
                                mse_hbm.at[pl.ds(PAD_OFF + done, n)])
                pltpu.sync_copy(wA.at[pl.ds(0, n)],
                                mde_hbm.at[pl.ds(PAD_OFF + done, n)])
                done += n


def _sc_scalers(src_p, dst_p):
    mesh = plsc.VectorSubcoreMesh(core_axis_name="c", subcore_axis_name="s",
                                  num_cores=NCORE, num_subcores=NSUB)
    kern = pl.kernel(
        _sca_body,
        out_type=[
            jax.ShapeDtypeStruct((ET,), jnp.int32),
            jax.ShapeDtypeStruct((ET,), jnp.int32),
            jax.ShapeDtypeStruct((ET,), jnp.int32),
            jax.ShapeDtypeStruct((ET,), jnp.float32),
        ],
        mesh=mesh,
        compiler_params=pltpu.CompilerParams(use_tc_tiling_on_sc=False,
                                             needs_layout_passes=False),
        scratch_types=[
            pltpu.VMEM((EPT1,), jnp.int32),
            pltpu.VMEM((EPT1,), jnp.int32),
            pltpu.VMEM((HS,), jnp.float32),
            pltpu.VMEM((WB,), jnp.int32),
            pltpu.VMEM((WB,), jnp.float32),
            pltpu.VMEM((HB,), jnp.float32),
            pltpu.VMEM((NSUB, HB), jnp.float32),
            pltpu.VMEM_SHARED((NSUB, HS), jnp.float32),
            pltpu.VMEM_SHARED((HS,), jnp.float32),
        ],
    )
    return kern(src_p, dst_p)


def _scb_body(se_hbm, mse_hbm, mde_hbm, s_hbm, stab_hbm, dtab_hbm,
              out_hbm,
              seS, mseS, mdeS, sS,
              srows, drows, msgB, idxS, idxD, sidx, seb, sb,
              acc, semGS, semGD, semW, semT):
    tid = lax.axis_index("s")
    cid = lax.axis_index("c")
    zero16 = jnp.zeros((16,), jnp.float32)
    base = tid * TPT

    def _zero_msg():
        def _zm(i, _):
            r = i // (Q // 16)
            c = (i % (Q // 16)) * 16
            msgB[0][r, pl.ds(c, 16)] = zero16
            return 0
        lax.fori_loop(0, C * (Q // 16), _zm, 0)

    def _zero_acc():
        def _za(nrows):
            off = 0
            while off < nrows:
                n = min(C, nrows - off)
                pltpu.sync_copy(msgB[0].at[pl.ds(0, n)],
                                acc.at[pl.ds(tid * ROWS_A + off, n)])
                off += n

        @pl.when(tid < NSUB - 1)
        def _():
            _za(ROWS_A)

        @pl.when(tid == NSUB - 1)
        def _():
            _za(ACC_R - (NSUB - 1) * ROWS_A)

    def _stage_fire(bi, par):
        boff = bi * (BS * C)
        pltpu.async_copy(se_hbm.at[pl.ds(base + boff, BS * C)], seS[par],
                         semT[par])
        pltpu.async_copy(mse_hbm.at[pl.ds(base + boff, BS * C)],
                         mseS[par], semT[par])
        pltpu.async_copy(mde_hbm.at[pl.ds(base + boff, BS * C)],
                         mdeS[par], semT[par])
        pltpu.async_copy(s_hbm.at[pl.ds(base + boff, BS * C)], sS[par],
                         semT[par])

    def _stage_wait(bi, par):
        boff = bi * (BS * C)
        pltpu.make_async_copy(se_hbm.at[pl.ds(base + boff, BS * C)],
                              seS[par], semT[par]).wait()
        pltpu.make_async_copy(mse_hbm.at[pl.ds(base + boff, BS * C)],
                              mseS[par], semT[par]).wait()
        pltpu.make_async_copy(mde_hbm.at[pl.ds(base + boff, BS * C)],
                              mdeS[par], semT[par]).wait()
        pltpu.make_async_copy(s_hbm.at[pl.ds(base + boff, BS * C)],
                              sS[par], semT[par]).wait()

    def _run_pass(p):
        qoff = (cid * 2 + p) * N

        def _fill_g(o, par, buf):
            def _f(k, _):
                j = o + k * 16
                idxS[buf][pl.ds(k * 16, 16)] = (
                    qoff + mseS[par][pl.ds(j, 16)])
                idxD[buf][pl.ds(k * 16, 16)] = (
                    qoff + mdeS[par][pl.ds(j, 16)])
                seb[buf][pl.ds(k * 16, 16)] = seS[par][pl.ds(j, 16)]
                sb[buf][pl.ds(k * 16, 16)] = sS[par][pl.ds(j, 16)]
                return 0
            lax.fori_loop(0, C // 16, _f, 0)

        def _fire_g(buf):
            pltpu.async_copy(stab_hbm.at[idxS[buf]], srows[buf],
                             semGS[buf])
            pltpu.async_copy(dtab_hbm.at[idxD[buf]], drows[buf],
                             semGD[buf])

        def _wait_g(buf):
            pltpu.make_async_copy(stab_hbm.at[idxS[buf]], srows[buf],
                                  semGS[buf]).wait()
            pltpu.make_async_copy(dtab_hbm.at[idxD[buf]], drows[buf],
                                  semGD[buf]).wait()

        def _wait_w(buf):
            pltpu.make_async_copy(msgB[buf], acc.at[sidx[buf]],
                                  semW[buf]).wait()

        def _compute(buf):
            ilv = plsc.PackFormat.INTERLEAVED

            def _one(e):
                s = plsc.load_gather(
                    sb[buf], [jnp.full((16,), e, jnp.int32)])
                for v in range(Q // 32):
                    a2 = srows[buf][e, pl.ds(v * 32, 32)]
                    hd2 = srows[buf][e, pl.ds(Q + v * 32, 32)]
                    b2 = drows[buf][e, pl.ds(v * 32, 32)]
                    he2 = drows[buf][e, pl.ds(Q + v * 32, 32)]
                    xw2 = drows[buf][e, pl.ds(2 * Q + v * 32, 32)]
                    av = plsc.unpack(a2, format=ilv)
                    hdv = plsc.unpack(hd2, format=ilv)
                    bv = plsc.unpack(b2, format=ilv)
                    hev = plsc.unpack(he2, format=ilv)
                    xwv = plsc.unpack(xw2, format=ilv)
                    for h in range(2):
                        g = (jnp.maximum(av[h] + bv[h], 0.0)
                             + hdv[h] + hev[h])
                        msgB[buf][e, pl.ds((2 * v + h) * 16, 16)] = (
                            xwv[h] * (s * g))

            def _pe(j, _):
                _one(2 * j)
                _one(2 * j + 1)
                return 0
            lax.fori_loop(0, C // 2, _pe, 0)

        def _fill_sidx(buf):
            def _f(k, _):
                sidx[buf][pl.ds(k * 16, 16)] = seb[buf][pl.ds(k * 16, 16)]
                return 0
            lax.fori_loop(0, C // 16, _f, 0)

        def _fire_w(buf):
            pltpu.async_copy(msgB[buf], acc.at[sidx[buf]], semW[buf],
                             add=True)

        _stage_fire(0, 0)
        _stage_wait(0, 0)
        _stage_fire(1, 1)
        _fill_g(0, 0, 0)
        _fire_g(0)
        _fill_g(C, 0, 1)
        _fire_g(1)

        def _block(b, _):
            for par in range(2):
                @pl.when(jnp.logical_and(b > 0, b % 2 == par))
                def _():
                    _stage_wait(b, par)

            def _pair(i, _):
                for buf in range(2):
                    _wait_g(buf)

                    @pl.when(jnp.logical_or(b > 0, i > 0))
                    def _():
                        _wait_w(buf)
                    _compute(buf)
                    _fill_sidx(buf)
                    _fire_w(buf)
                    last = BS // 2 - 1
                    for par in range(2):
                        @pl.when(jnp.logical_and(i < last, b % 2 == par))
                        def _():
                            _fill_g((2 * i + buf + 2) * C, par, buf)
                            _fire_g(buf)

                        @pl.when(jnp.logical_and(
                            i == last,
                            jnp.logical_and(b + 1 < NBLK,
                                            (b + 1) % 2 == par)))
                        def _():
                            _fill_g(buf * C, par, buf)
                            _fire_g(buf)
                return 0
            lax.fori_loop(0, BS // 2, _pair, 0)

            for par in range(2):
                @pl.when(jnp.logical_and(b + 2 < NBLK, b % 2 == par))
                def _():
                    _stage_fire(b + 2, par)
            return 0
        lax.fori_loop(0, NBLK, _block, 0)

        _wait_w(0)
        _wait_w(1)
        plsc.subcore_barrier()

        def _co(nrows):
            off = 0
            while off < nrows:
                n = min(C, nrows - off)
                pltpu.sync_copy(acc.at[pl.ds(tid * ROWS_A + off, n)],
                                msgB[0].at[pl.ds(0, n)])
                pltpu.sync_copy(
                    msgB[0].at[pl.ds(0, n)],
                    out_hbm.at[pl.ds(qoff + tid * ROWS_A + off, n)])
                off += n

        @pl.when(tid < NSUB - 1)
        def _():
            _co(ROWS_A)

        @pl.when(tid == NSUB - 1)
        def _():
            _co(N - (NSUB - 1) * ROWS_A)
        plsc.subcore_barrier()

    for p in range(2):
        _zero_msg()
        _zero_acc()
        plsc.subcore_barrier()
        _run_pass(p)


def _sc_aggregate(se, mse, mde, s, stab4, dtab4):
    mesh = plsc.VectorSubcoreMesh(core_axis_name="c", subcore_axis_name="s",
                                  num_cores=NCORE, num_subcores=NSUB)
    kern = pl.kernel(
        _scb_body,
        out_type=jax.ShapeDtypeStruct((4 * N, Q), jnp.float32),
        mesh=mesh,
        compiler_params=pltpu.CompilerParams(use_tc_tiling_on_sc=False,
                                             needs_layout_passes=False),
        scratch_types=[
            [pltpu.VMEM((BS * C,), jnp.int32)] * 2,
            [pltpu.VMEM((BS * C,), jnp.int32)] * 2,
            [pltpu.VMEM((BS * C,), jnp.int32)] * 2,
            [pltpu.VMEM((BS * C,), jnp.float32)] * 2,
            [pltpu.VMEM((C, 2 * Q), jnp.bfloat16)] * 2,
            [pltpu.VMEM((C, 3 * Q), jnp.bfloat16)] * 2,
            [pltpu.VMEM((C, Q), jnp.float32)] * 2,
            [pltpu.VMEM((C,), jnp.int32)] * 2,
            [pltpu.VMEM((C,), jnp.int32)] * 2,
            [pltpu.VMEM((C,), jnp.int32)] * 2,
            [pltpu.VMEM((C,), jnp.int32)] * 2,
            [pltpu.VMEM((C,), jnp.float32)] * 2,
            pltpu.VMEM_SHARED((ACC_R, Q), jnp.float32),
            [pltpu.SemaphoreType.DMA] * 2,
            [pltpu.SemaphoreType.DMA] * 2,
            [pltpu.SemaphoreType.DMA] * 2,
            [pltpu.SemaphoreType.DMA] * 2,
        ],
    )
    return kern(se, mse, mde, s, stab4, dtab4)


def kernel(x, edge_index, W_w, W_b, U_w, U_b, V_w, V_b, D_w, D_b, E_w, E_b):
    d = x.shape[1]

    ut = U_w.T
    wt = W_w.T
    wcat = jnp.concatenate(
        [V_w[:, :d].T, V_w[:, d:].T, D_w.T, E_w.T], axis=1)
    bcat = jnp.concatenate(
        [jnp.zeros((d,), jnp.float32), V_b, D_b, E_b]).reshape(1, 4 * d)
    ub = U_b.reshape(1, d)
    wb = W_b.reshape(1, d)

    stab, dtab = _tc_tables(x, ut, ub, wcat, bcat, wt, wb)
    stab4 = stab.reshape(4 * N, 2 * Q).astype(jnp.bfloat16)
    dtab4 = dtab.reshape(4 * N, 3 * Q).astype(jnp.bfloat16)

    src_p = edge_index[0].astype(jnp.int32)
    dst_p = edge_index[1].astype(jnp.int32)

    se, mse, mde, s = _sc_scalers(src_p, dst_p)
    out4 = _sc_aggregate(se, mse, mde, s, stab4, dtab4)
    perm = [0] * Q
    for v in range(Q // 32):
        for h in range(2):
            for k in range(16):
                perm[(2 * v + h) * 16 + k] = 32 * v + 2 * k + h
    inv = [0] * Q
    for j, c in enumerate(perm):
        inv[c] = j
    out4 = out4[:, jnp.array(inv, dtype=jnp.int32)]
    out = out4.reshape(4, N, Q).transpose(1, 0, 2).reshape(N, d)
    return out


# speedup vs baseline: 5.4245x; 1.0006x over previous
import functools

import jax
import jax.numpy as jnp
from jax import lax
from jax.experimental import pallas as pl
from jax.experimental.pallas import tpu as pltpu
from jax.experimental.pallas import tpu_sc as plsc

N = 10000
D = 256
Q = D // 4
SENT = N
NSUB = 16
NCORE = 2
C = 128
HS = 10240
HB = HS // NSUB
ACC_R = N + NSUB
WB = 2000
BS = 6
E_IN = 160000
EPT1 = E_IN // NSUB
ET = 172032
TPT = ET // NSUB
NCH = TPT // C
NBLK = NCH // BS
PAD_OFF = E_IN + N
ROWS_A = 632


def _rsqrt_newton(xx):
    bits = plsc.bitcast(xx, jnp.int32)
    y = plsc.bitcast(
        jnp.int32(0x5F3759DF) - lax.shift_right_logical(bits, 1),
        jnp.float32)
    for _ in range(3):
        y = y * (1.5 - 0.5 * xx * y * y)
    return y


def _tc_body(x_ref, ut_ref, ub_ref, wcat_ref, bcat_ref, wt_ref, wb_ref,
             stab_ref, dtab_ref):
    xb = x_ref[...].astype(jnp.bfloat16)
    h = jnp.dot(xb, ut_ref[...].astype(jnp.bfloat16),
                preferred_element_type=jnp.float32)
    h = h + ub_ref[...]
    y = jnp.dot(h.astype(jnp.bfloat16),
                wcat_ref[...].astype(jnp.bfloat16),
                preferred_element_type=jnp.float32)
    y = y + bcat_ref[...]
    xw = jnp.dot(xb, wt_ref[...].astype(jnp.bfloat16),
                 preferred_element_type=jnp.float32)
    xw = xw + wb_ref[...]
    a = y[:, 0:D]
    b = y[:, D:2 * D]
    hd = y[:, 2 * D:3 * D]
    he = y[:, 3 * D:4 * D]
    stab_ref[...] = jnp.stack(
        [jnp.concatenate([a[:, q * Q:(q + 1) * Q],
                          hd[:, q * Q:(q + 1) * Q]], axis=1)
         for q in range(4)], axis=0)
    dtab_ref[...] = jnp.stack(
        [jnp.concatenate([b[:, q * Q:(q + 1) * Q],
                          he[:, q * Q:(q + 1) * Q],
                          xw[:, q * Q:(q + 1) * Q]], axis=1)
         for q in range(4)], axis=0)


def _tc_tables(x, ut, ub, wcat, bcat, wt, wb):
    nb = 10
    blk = N // nb
    return pl.pallas_call(
        _tc_body,
        grid=(nb,),
        in_specs=[
            pl.BlockSpec((blk, D), lambda i: (i, 0)),
            pl.BlockSpec((D, D), lambda i: (0, 0)),
            pl.BlockSpec((1, D), lambda i: (0, 0)),
            pl.BlockSpec((D, 4 * D), lambda i: (0, 0)),
            pl.BlockSpec((1, 4 * D), lambda i: (0, 0)),
            pl.BlockSpec((D, D), lambda i: (0, 0)),
            pl.BlockSpec((1, D), lambda i: (0, 0)),
        ],
        out_specs=[
            pl.BlockSpec((4, blk, 2 * Q), lambda i: (0, i, 0)),
            pl.BlockSpec((4, blk, 3 * Q), lambda i: (0, i, 0)),
        ],
        out_shape=[
            jax.ShapeDtypeStruct((4, N, 2 * Q), jnp.float32),
            jax.ShapeDtypeStruct((4, N, 3 * Q), jnp.float32),
        ],
    )(x, ut, ub, wcat, bcat, wt, wb)


def _sca_body(src_hbm, dst_hbm,
              se_hbm, mse_hbm, mde_hbm, s_hbm,
              srcT, dstT, histL, wA, wC, ssum, tmpv,
              staging, histG):
    tid = lax.axis_index("s")
    cid = lax.axis_index("c")
    zero16 = jnp.zeros((16,), jnp.float32)

    base = tid * EPT1
    pltpu.sync_copy(src_hbm.at[pl.ds(base, EPT1)], srcT)
    pltpu.sync_copy(dst_hbm.at[pl.ds(base, EPT1)], dstT)

    def _zh(i, _):
        histL[pl.ds(i * 16, 16)] = zero16
        return 0
    lax.fori_loop(0, HS // 16, _zh, 0)

    def _zs(i, _):
        ssum[pl.ds(i * 16, 16)] = zero16
        return 0
    lax.fori_loop(0, HB // 16, _zs, 0)

    ones16 = jnp.ones((16,), jnp.float32)

    def _blk_a(bi, _):
        def _grp(k, _):
            j = bi * WB + k * 16
            sv = srcT[pl.ds(j, 16)]
            dv = dstT[pl.ds(j, 16)]
            m = sv != dv
            se = jnp.where(m, sv, SENT)
            plsc.addupdate_scatter(histL, [se], ones16)
            wA[pl.ds(k * 16, 16)] = se
            return 0
        lax.fori_loop(0, WB // 16, _grp, 0)

        @pl.when(cid == 0)
        def _():
            pltpu.sync_copy(wA, se_hbm.at[pl.ds(base + bi * WB, WB)])

        def _grp2(k, _):
            j = bi * WB + k * 16
            sv = srcT[pl.ds(j, 16)]
            dv = dstT[pl.ds(j, 16)]
            m = sv != dv
            wA[pl.ds(k * 16, 16)] = jnp.minimum(
                jnp.where(m, sv, SENT), N - 1)
            return 0
        lax.fori_loop(0, WB // 16, _grp2, 0)

        @pl.when(cid == 0)
        def _():
            pltpu.sync_copy(wA, mse_hbm.at[pl.ds(base + bi * WB, WB)])

        def _grp3(k, _):
            j = bi * WB + k * 16
            sv = srcT[pl.ds(j, 16)]
            dv = dstT[pl.ds(j, 16)]
            m = sv != dv
            wA[pl.ds(k * 16, 16)] = jnp.minimum(
                jnp.where(m, dv, SENT), N - 1)
            return 0
        lax.fori_loop(0, WB // 16, _grp3, 0)

        @pl.when(cid == 0)
        def _():
            pltpu.sync_copy(wA, mde_hbm.at[pl.ds(base + bi * WB, WB)])
        return 0
    lax.fori_loop(0, EPT1 // WB, _blk_a, 0)

    pltpu.sync_copy(histL, staging.at[tid])
    plsc.subcore_barrier()

    pltpu.sync_copy(staging.at[:, pl.ds(tid * HB, HB)], tmpv)

    def _red(j, _):
        def _acc(v, _):
            ssum[pl.ds(v * 16, 16)] = (ssum[pl.ds(v * 16, 16)]
                                       + tmpv[j, pl.ds(v * 16, 16)])
            return 0
        lax.fori_loop(0, HB // 16, _acc, 0)
        return 0
    lax.fori_loop(0, NSUB, _red, 0)
    pltpu.sync_copy(ssum, histG.at[pl.ds(tid * HB, HB)])
    plsc.subcore_barrier()

    pltpu.sync_copy(histG, histL)

    def _rsq(i, _):
        h = histL[pl.ds(i * 16, 16)]
        idx = lax.iota(jnp.int32, 16) + i * 16
        valid = idx < N
        deg = h + jnp.where(valid, 1.0, 0.0)
        y = _rsqrt_newton(jnp.maximum(deg, 1.0))
        histL[pl.ds(i * 16, 16)] = jnp.where(valid, y, 0.0)
        return 0
    lax.fori_loop(0, HS // 16, _rsq, 0)

    def _blk_b(bi, _):
        def _grp(k, _):
            j = bi * WB + k * 16
            sv = srcT[pl.ds(j, 16)]
            dv = dstT[pl.ds(j, 16)]
            m = sv != dv
            se = jnp.where(m, sv, SENT)
            de = jnp.where(m, dv, SENT)
            rs = plsc.load_gather(histL, [se])
            rd = plsc.load_gather(histL, [de])
            wC[pl.ds(k * 16, 16)] = rs * rd
            return 0
        lax.fori_loop(0, WB // 16, _grp, 0)

        @pl.when(cid == 0)
        def _():
            pltpu.sync_copy(wC, s_hbm.at[pl.ds(base + bi * WB, WB)])
        return 0
    lax.fori_loop(0, EPT1 // WB, _blk_b, 0)

    @pl.when(cid == 0)
    def _():
        nrows = jnp.where(tid < NSUB - 1, 0, 0)
        del nrows

        def _self(nrows):
            nch = -(-nrows // 16)

            def _g(k, _):
                node = tid * ROWS_A + k * 16 + lax.iota(jnp.int32, 16)
                node = jnp.minimum(node, N - 1)
                wA[pl.ds(k * 16, 16)] = node
                r = plsc.load_gather(histL, [node])
                wC[pl.ds(k * 16, 16)] = r * r
                return 0
            lax.fori_loop(0, nch, _g, 0)
            off = E_IN + tid * ROWS_A
            pltpu.sync_copy(wA.at[pl.ds(0, nrows)],
                            se_hbm.at[pl.ds(off, nrows)])
            pltpu.sync_copy(wA.at[pl.ds(0, nrows)],
                            mse_hbm.at[pl.ds(off, nrows)])
            pltpu.sync_copy(wA.at[pl.ds(0, nrows)],
                            mde_hbm.at[pl.ds(off, nrows)])
            pltpu.sync_copy(wC.at[pl.ds(0, nrows)],
                            s_hbm.at[pl.ds(off, nrows)])

        @pl.when(tid < NSUB - 1)
        def _():
            _self(ROWS_A)

        @pl.when(tid == NSUB - 1)
        def _():
            _self(N - (NSUB - 1) * ROWS_A)
            npad = ET - PAD_OFF

            def _gp(k, _):
                wA[pl.ds(k * 16, 16)] = jnp.full((16,), SENT, jnp.int32)
                wC[pl.ds(k * 16, 16)] = jnp.zeros((16,), jnp.float32)
                return 0
            lax.fori_loop(0, WB // 16, _gp, 0)
            done = 0
            while done < npad:
                n = min(WB, npad - done)
                pltpu.sync_copy(wA.at[pl.ds(0, n)],
                                se_hbm.at[pl.ds(PAD_OFF + done, n)])
                pltpu.sync_copy(wC.at[pl.ds(0, n)],
                                s_hbm.at[pl.ds(PAD_OFF + done, n)])
                done += n

            def _gq(k, _):
                wA[pl.ds(k * 16, 16)] = jnp.full((16,), N - 1, jnp.int32)
                return 0
            lax.fori_loop(0, WB // 16, _gq, 0)
            done = 0
            while done < npad:
                n = min(WB, npad - done)
                pltpu.sync_copy(wA.at[pl.ds(0, n)],
                                mse_hbm.at[pl.ds(PAD_OFF + done, n)])
                pltpu.sync_copy(wA.at[pl.ds(0, n)],
                                mde_hbm.at[pl.ds(PAD_OFF + done, n)])
                done += n


def _sc_scalers(src_p, dst_p):
    mesh = plsc.VectorSubcoreMesh(core_axis_name="c", subcore_axis_name="s",
                                  num_cores=NCORE, num_subcores=NSUB)
    kern = pl.kernel(
        _sca_body,
        out_type=[
            jax.ShapeDtypeStruct((ET,), jnp.int32),
            jax.ShapeDtypeStruct((ET,), jnp.int32),
            jax.ShapeDtypeStruct((ET,), jnp.int32),
            jax.ShapeDtypeStruct((ET,), jnp.float32),
        ],
        mesh=mesh,
        compiler_params=pltpu.CompilerParams(use_tc_tiling_on_sc=False,
                                             needs_layout_passes=False),
        scratch_types=[
            pltpu.VMEM((EPT1,), jnp.int32),
            pltpu.VMEM((EPT1,), jnp.int32),
            pltpu.VMEM((HS,), jnp.float32),
            pltpu.VMEM((WB,), jnp.int32),
            pltpu.VMEM((WB,), jnp.float32),
            pltpu.VMEM((HB,), jnp.float32),
            pltpu.VMEM((NSUB, HB), jnp.float32),
            pltpu.VMEM_SHARED((NSUB, HS), jnp.float32),
            pltpu.VMEM_SHARED((HS,), jnp.float32),
        ],
    )
    return kern(src_p, dst_p)


def _scb_body(se_hbm, mse_hbm, mde_hbm, s_hbm, stab_hbm, dtab_hbm,
              out_hbm,
              seS, mseS, mdeS, sS,
              srows, drows, msgB, idxS, idxD, sidx, seb, sb,
              acc, semGS, semGD, semW, semT):
    tid = lax.axis_index("s")
    cid = lax.axis_index("c")
    zero16 = jnp.zeros((16,), jnp.float32)
    base = tid * TPT

    def _zero_msg():
        def _zm(i, _):
            r = i // (Q // 16)
            c = (i % (Q // 16)) * 16
            msgB[0][r, pl.ds(c, 16)] = zero16
            return 0
        lax.fori_loop(0, C * (Q // 16), _zm, 0)

    def _zero_acc():
        def _za(nrows):
            off = 0
            while off < nrows:
                n = min(C, nrows - off)
                pltpu.sync_copy(msgB[0].at[pl.ds(0, n)],
                                acc.at[pl.ds(tid * ROWS_A + off, n)])
                off += n

        @pl.when(tid < NSUB - 1)
        def _():
            _za(ROWS_A)

        @pl.when(tid == NSUB - 1)
        def _():
            _za(ACC_R - (NSUB - 1) * ROWS_A)

    def _stage_fire(bi, par):
        boff = bi * (BS * C)
        pltpu.async_copy(se_hbm.at[pl.ds(base + boff, BS * C)], seS[par],
                         semT[par])
        pltpu.async_copy(mse_hbm.at[pl.ds(base + boff, BS * C)],
                         mseS[par], semT[par])
        pltpu.async_copy(mde_hbm.at[pl.ds(base + boff, BS * C)],
                         mdeS[par], semT[par])
        pltpu.async_copy(s_hbm.at[pl.ds(base + boff, BS * C)], sS[par],
                         semT[par])

    def _stage_wait(bi, par):
        boff = bi * (BS * C)
        pltpu.make_async_copy(se_hbm.at[pl.ds(base + boff, BS * C)],
                              seS[par], semT[par]).wait()
        pltpu.make_async_copy(mse_hbm.at[pl.ds(base + boff, BS * C)],
                              mseS[par], semT[par]).wait()
        pltpu.make_async_copy(mde_hbm.at[pl.ds(base + boff, BS * C)],
                              mdeS[par], semT[par]).wait()
        pltpu.make_async_copy(s_hbm.at[pl.ds(base + boff, BS * C)],
                              sS[par], semT[par]).wait()

    def _run_pass(p):
        qoff = (cid * 2 + p) * N

        def _fill_g(o, par, buf):
            def _f(k, _):
                j = o + k * 16
                idxS[buf][pl.ds(k * 16, 16)] = (
                    qoff + mseS[par][pl.ds(j, 16)])
                idxD[buf][pl.ds(k * 16, 16)] = (
                    qoff + mdeS[par][pl.ds(j, 16)])
                seb[buf][pl.ds(k * 16, 16)] = seS[par][pl.ds(j, 16)]
                sb[buf][pl.ds(k * 16, 16)] = sS[par][pl.ds(j, 16)]
                return 0
            lax.fori_loop(0, C // 16, _f, 0)

        def _fire_g(buf):
            pltpu.async_copy(stab_hbm.at[idxS[buf]], srows[buf],
                             semGS[buf])
            pltpu.async_copy(dtab_hbm.at[idxD[buf]], drows[buf],
                             semGD[buf])

        def _wait_g(buf):
            pltpu.make_async_copy(stab_hbm.at[idxS[buf]], srows[buf],
                                  semGS[buf]).wait()
            pltpu.make_async_copy(dtab_hbm.at[idxD[buf]], drows[buf],
                                  semGD[buf]).wait()

        def _wait_w(buf):
            pltpu.make_async_copy(msgB[buf], acc.at[sidx[buf]],
                                  semW[buf]).wait()

        def _compute(buf):
            ilv = plsc.PackFormat.INTERLEAVED

            def _one(e):
                s = plsc.load_gather(
                    sb[buf], [jnp.full((16,), e, jnp.int32)])
                for v in range(Q // 32):
                    a2 = srows[buf][e, pl.ds(v * 32, 32)]
                    hd2 = srows[buf][e, pl.ds(Q + v * 32, 32)]
                    b2 = drows[buf][e, pl.ds(v * 32, 32)]
                    he2 = drows[buf][e, pl.ds(Q + v * 32, 32)]
                    xw2 = drows[buf][e, pl.ds(2 * Q + v * 32, 32)]
                    av = plsc.unpack(a2, format=ilv)
                    hdv = plsc.unpack(hd2, format=ilv)
                    bv = plsc.unpack(b2, format=ilv)
                    hev = plsc.unpack(he2, format=ilv)
                    xwv = plsc.unpack(xw2, format=ilv)
                    for h in range(2):
                        g = (jnp.maximum(av[h] + bv[h], 0.0)
                             + hdv[h] + hev[h])
                        msgB[buf][e, pl.ds((2 * v + h) * 16, 16)] = (
                            xwv[h] * (s * g))

            def _pe(j, _):
                _one(2 * j)
                _one(2 * j + 1)
                return 0
            lax.fori_loop(0, C // 2, _pe, 0)

        def _fill_sidx(buf):
            def _f(k, _):
                sidx[buf][pl.ds(k * 16, 16)] = seb[buf][pl.ds(k * 16, 16)]
                return 0
            lax.fori_loop(0, C // 16, _f, 0)

        def _fire_w(buf):
            pltpu.async_copy(msgB[buf], acc.at[sidx[buf]], semW[buf],
                             add=True)

        _stage_fire(0, 0)
        _stage_wait(0, 0)
        _stage_fire(1, 1)
        _fill_g(0, 0, 0)
        _fire_g(0)
        _fill_g(C, 0, 1)
        _fire_g(1)

        def _block(b, _):
            for par in range(2):
                @pl.when(jnp.logical_and(b > 0, b % 2 == par))
                def _():
                    _stage_wait(b, par)

            def _pair(i, _):
                for buf in range(2):
                    _wait_g(buf)

                    @pl.when(jnp.logical_or(b > 0, i > 0))
                    def _():
                        _wait_w(buf)
                    _compute(buf)
                    _fill_sidx(buf)
                    _fire_w(buf)
                    last = BS // 2 - 1
                    for par in range(2):
                        @pl.when(jnp.logical_and(i < last, b % 2 == par))
                        def _():
                            _fill_g((2 * i + buf + 2) * C, par, buf)
                            _fire_g(buf)

                        @pl.when(jnp.logical_and(
                            i == last,
                            jnp.logical_and(b + 1 < NBLK,
                                            (b + 1) % 2 == par)))
                        def _():
                            _fill_g(buf * C, par, buf)
                            _fire_g(buf)
                return 0
            lax.fori_loop(0, BS // 2, _pair, 0)

            for par in range(2):
                @pl.when(jnp.logical_and(b + 2 < NBLK, b % 2 == par))
                def _():
                    _stage_fire(b + 2, par)
            return 0
        lax.fori_loop(0, NBLK, _block, 0)

        _wait_w(0)
        _wait_w(1)
        plsc.subcore_barrier()

        def _co(nrows):
            off = 0
            while off < nrows:
                n = min(C, nrows - off)
                pltpu.sync_copy(acc.at[pl.ds(tid * ROWS_A + off, n)],
                                msgB[0].at[pl.ds(0, n)])
                pltpu.sync_copy(
                    msgB[0].at[pl.ds(0, n)],
                    out_hbm.at[pl.ds(qoff + tid * ROWS_A + off, n)])
                off += n

        @pl.when(tid < NSUB - 1)
        def _():
            _co(ROWS_A)

        @pl.when(tid == NSUB - 1)
        def _():
            _co(N - (NSUB - 1) * ROWS_A)
        plsc.subcore_barrier()

    for p in range(2):
        _zero_msg()
        _zero_acc()
        plsc.subcore_barrier()
        _run_pass(p)


def _sc_aggregate(se, mse, mde, s, stab4, dtab4):
    mesh = plsc.VectorSubcoreMesh(core_axis_name="c", subcore_axis_name="s",
                                  num_cores=NCORE, num_subcores=NSUB)
    kern = pl.kernel(
        _scb_body,
        out_type=jax.ShapeDtypeStruct((4 * N, Q), jnp.float32),
        mesh=mesh,
        compiler_params=pltpu.CompilerParams(use_tc_tiling_on_sc=False,
                                             needs_layout_passes=False),
        scratch_types=[
            [pltpu.VMEM((BS * C,), jnp.int32)] * 2,
            [pltpu.VMEM((BS * C,), jnp.int32)] * 2,
            [pltpu.VMEM((BS * C,), jnp.int32)] * 2,
            [pltpu.VMEM((BS * C,), jnp.float32)] * 2,
            [pltpu.VMEM((C, 2 * Q), jnp.bfloat16)] * 2,
            [pltpu.VMEM((C, 3 * Q), jnp.bfloat16)] * 2,
            [pltpu.VMEM((C, Q), jnp.float32)] * 2,
            [pltpu.VMEM((C,), jnp.int32)] * 2,
            [pltpu.VMEM((C,), jnp.int32)] * 2,
            [pltpu.VMEM((C,), jnp.int32)] * 2,
            [pltpu.VMEM((C,), jnp.int32)] * 2,
            [pltpu.VMEM((C,), jnp.float32)] * 2,
            pltpu.VMEM_SHARED((ACC_R, Q), jnp.float32),
            [pltpu.SemaphoreType.DMA] * 2,
            [pltpu.SemaphoreType.DMA] * 2,
            [pltpu.SemaphoreType.DMA] * 2,
            [pltpu.SemaphoreType.DMA] * 2,
        ],
    )
    return kern(se, mse, mde, s, stab4, dtab4)


def kernel(x, edge_index, W_w, W_b, U_w, U_b, V_w, V_b, D_w, D_b, E_w, E_b):
    d = x.shape[1]

    ut = U_w.T
    wt = W_w.T
    wcat = jnp.concatenate(
        [V_w[:, :d].T, V_w[:, d:].T, D_w.T, E_w.T], axis=1)
    bcat = jnp.concatenate(
        [jnp.zeros((d,), jnp.float32), V_b, D_b, E_b]).reshape(1, 4 * d)
    ub = U_b.reshape(1, d)
    wb = W_b.reshape(1, d)

    stab, dtab = _tc_tables(x, ut, ub, wcat, bcat, wt, wb)
    stab4 = stab.reshape(4 * N, 2 * Q).astype(jnp.bfloat16)
    dtab4 = dtab.reshape(4 * N, 3 * Q).astype(jnp.bfloat16)

    src_p = edge_index[0].astype(jnp.int32)
    dst_p = edge_index[1].astype(jnp.int32)

    se, mse, mde, s = _sc_scalers(src_p, dst_p)
    out4 = _sc_aggregate(se, mse, mde, s, stab4, dtab4)
    perm = [0] * Q
    for v in range(Q // 32):
        for h in range(2):
            for k in range(16):
                perm[(2 * v + h) * 16 + k] = 32 * v + 2 * k + h
    inv = [0] * Q
    for j, c in enumerate(perm):
        inv[c] = j
    out4 = out4[:, jnp.array(inv, dtype=jnp.int32)]
    out = out4.reshape(4, N, Q).transpose(1, 0, 2).reshape(N, d)
    return out


# speedup vs baseline: 5.4529x; 1.0052x over previous
import functools

import jax
import jax.numpy as jnp
from jax import lax
from jax.experimental import pallas as pl
from jax.experimental.pallas import tpu as pltpu
from jax.experimental.pallas import tpu_sc as plsc

N = 10000
D = 256
Q = D // 4
SENT = N
NSUB = 16
NCORE = 2
C = 112
HS = 10240
HB = HS // NSUB
ACC_R = N + NSUB
WB = 2000
BS = 6
E_IN = 160000
EPT1 = E_IN // NSUB
ET = 172032
TPT = ET // NSUB
NCH = TPT // C
NBLK = NCH // BS
PAD_OFF = E_IN + N
ROWS_A = 632


def _rsqrt_newton(xx):
    bits = plsc.bitcast(xx, jnp.int32)
    y = plsc.bitcast(
        jnp.int32(0x5F3759DF) - lax.shift_right_logical(bits, 1),
        jnp.float32)
    for _ in range(3):
        y = y * (1.5 - 0.5 * xx * y * y)
    return y


def _tc_body(x_ref, ut_ref, ub_ref, wcat_ref, bcat_ref, wt_ref, wb_ref,
             stab_ref, dtab_ref):
    xb = x_ref[...].astype(jnp.bfloat16)
    h = jnp.dot(xb, ut_ref[...].astype(jnp.bfloat16),
                preferred_element_type=jnp.float32)
    h = h + ub_ref[...]
    y = jnp.dot(h.astype(jnp.bfloat16),
                wcat_ref[...].astype(jnp.bfloat16),
                preferred_element_type=jnp.float32)
    y = y + bcat_ref[...]
    xw = jnp.dot(xb, wt_ref[...].astype(jnp.bfloat16),
                 preferred_element_type=jnp.float32)
    xw = xw + wb_ref[...]
    a = y[:, 0:D]
    b = y[:, D:2 * D]
    hd = y[:, 2 * D:3 * D]
    he = y[:, 3 * D:4 * D]
    stab_ref[...] = jnp.stack(
        [jnp.concatenate([a[:, q * Q:(q + 1) * Q],
                          hd[:, q * Q:(q + 1) * Q]], axis=1)
         for q in range(4)], axis=0)
    dtab_ref[...] = jnp.stack(
        [jnp.concatenate([b[:, q * Q:(q + 1) * Q],
                          he[:, q * Q:(q + 1) * Q],
                          xw[:, q * Q:(q + 1) * Q]], axis=1)
         for q in range(4)], axis=0)


def _tc_tables(x, ut, ub, wcat, bcat, wt, wb):
    nb = 10
    blk = N // nb
    return pl.pallas_call(
        _tc_body,
        grid=(nb,),
        in_specs=[
            pl.BlockSpec((blk, D), lambda i: (i, 0)),
            pl.BlockSpec((D, D), lambda i: (0, 0)),
            pl.BlockSpec((1, D), lambda i: (0, 0)),
            pl.BlockSpec((D, 4 * D), lambda i: (0, 0)),
            pl.BlockSpec((1, 4 * D), lambda i: (0, 0)),
            pl.BlockSpec((D, D), lambda i: (0, 0)),
            pl.BlockSpec((1, D), lambda i: (0, 0)),
        ],
        out_specs=[
            pl.BlockSpec((4, blk, 2 * Q), lambda i: (0, i, 0)),
            pl.BlockSpec((4, blk, 3 * Q), lambda i: (0, i, 0)),
        ],
        out_shape=[
            jax.ShapeDtypeStruct((4, N, 2 * Q), jnp.float32),
            jax.ShapeDtypeStruct((4, N, 3 * Q), jnp.float32),
        ],
    )(x, ut, ub, wcat, bcat, wt, wb)


def _sca_body(src_hbm, dst_hbm,
              se_hbm, mse_hbm, mde_hbm, s_hbm,
              srcT, dstT, histL, wA, wC, ssum, tmpv,
              staging, histG):
    tid = lax.axis_index("s")
    cid = lax.axis_index("c")
    zero16 = jnp.zeros((16,), jnp.float32)

    base = tid * EPT1
    pltpu.sync_copy(src_hbm.at[pl.ds(base, EPT1)], srcT)
    pltpu.sync_copy(dst_hbm.at[pl.ds(base, EPT1)], dstT)

    def _zh(i, _):
        histL[pl.ds(i * 16, 16)] = zero16
        return 0
    lax.fori_loop(0, HS // 16, _zh, 0)

    def _zs(i, _):
        ssum[pl.ds(i * 16, 16)] = zero16
        return 0
    lax.fori_loop(0, HB // 16, _zs, 0)

    ones16 = jnp.ones((16,), jnp.float32)

    def _blk_a(bi, _):
        def _grp(k, _):
            j = bi * WB + k * 16
            sv = srcT[pl.ds(j, 16)]
            dv = dstT[pl.ds(j, 16)]
            m = sv != dv
            se = jnp.where(m, sv, SENT)
            plsc.addupdate_scatter(histL, [se], ones16)
            wA[pl.ds(k * 16, 16)] = se
            return 0
        lax.fori_loop(0, WB // 16, _grp, 0)

        @pl.when(cid == 0)
        def _():
            pltpu.sync_copy(wA, se_hbm.at[pl.ds(base + bi * WB, WB)])

        def _grp2(k, _):
            j = bi * WB + k * 16
            sv = srcT[pl.ds(j, 16)]
            dv = dstT[pl.ds(j, 16)]
            m = sv != dv
            wA[pl.ds(k * 16, 16)] = jnp.minimum(
                jnp.where(m, sv, SENT), N - 1)
            return 0
        lax.fori_loop(0, WB // 16, _grp2, 0)

        @pl.when(cid == 0)
        def _():
            pltpu.sync_copy(wA, mse_hbm.at[pl.ds(base + bi * WB, WB)])

        def _grp3(k, _):
            j = bi * WB + k * 16
            sv = srcT[pl.ds(j, 16)]
            dv = dstT[pl.ds(j, 16)]
            m = sv != dv
            wA[pl.ds(k * 16, 16)] = jnp.minimum(
                jnp.where(m, dv, SENT), N - 1)
            return 0
        lax.fori_loop(0, WB // 16, _grp3, 0)

        @pl.when(cid == 0)
        def _():
            pltpu.sync_copy(wA, mde_hbm.at[pl.ds(base + bi * WB, WB)])
        return 0
    lax.fori_loop(0, EPT1 // WB, _blk_a, 0)

    pltpu.sync_copy(histL, staging.at[tid])
    plsc.subcore_barrier()

    pltpu.sync_copy(staging.at[:, pl.ds(tid * HB, HB)], tmpv)

    def _red(j, _):
        def _acc(v, _):
            ssum[pl.ds(v * 16, 16)] = (ssum[pl.ds(v * 16, 16)]
                                       + tmpv[j, pl.ds(v * 16, 16)])
            return 0
        lax.fori_loop(0, HB // 16, _acc, 0)
        return 0
    lax.fori_loop(0, NSUB, _red, 0)
    pltpu.sync_copy(ssum, histG.at[pl.ds(tid * HB, HB)])
    plsc.subcore_barrier()

    pltpu.sync_copy(histG, histL)

    def _rsq(i, _):
        h = histL[pl.ds(i * 16, 16)]
        idx = lax.iota(jnp.int32, 16) + i * 16
        valid = idx < N
        deg = h + jnp.where(valid, 1.0, 0.0)
        y = _rsqrt_newton(jnp.maximum(deg, 1.0))
        histL[pl.ds(i * 16, 16)] = jnp.where(valid, y, 0.0)
        return 0
    lax.fori_loop(0, HS // 16, _rsq, 0)

    def _blk_b(bi, _):
        def _grp(k, _):
            j = bi * WB + k * 16
            sv = srcT[pl.ds(j, 16)]
            dv = dstT[pl.ds(j, 16)]
            m = sv != dv
            se = jnp.where(m, sv, SENT)
            de = jnp.where(m, dv, SENT)
            rs = plsc.load_gather(histL, [se])
            rd = plsc.load_gather(histL, [de])
            wC[pl.ds(k * 16, 16)] = rs * rd
            return 0
        lax.fori_loop(0, WB // 16, _grp, 0)

        @pl.when(cid == 0)
        def _():
            pltpu.sync_copy(wC, s_hbm.at[pl.ds(base + bi * WB, WB)])
        return 0
    lax.fori_loop(0, EPT1 // WB, _blk_b, 0)

    @pl.when(cid == 0)
    def _():
        nrows = jnp.where(tid < NSUB - 1, 0, 0)
        del nrows

        def _self(nrows):
            nch = -(-nrows // 16)

            def _g(k, _):
                node = tid * ROWS_A + k * 16 + lax.iota(jnp.int32, 16)
                node = jnp.minimum(node, N - 1)
                wA[pl.ds(k * 16, 16)] = node
                r = plsc.load_gather(histL, [node])
                wC[pl.ds(k * 16, 16)] = r * r
                return 0
            lax.fori_loop(0, nch, _g, 0)
            off = E_IN + tid * ROWS_A
            pltpu.sync_copy(wA.at[pl.ds(0, nrows)],
                            se_hbm.at[pl.ds(off, nrows)])
            pltpu.sync_copy(wA.at[pl.ds(0, nrows)],
                            mse_hbm.at[pl.ds(off, nrows)])
            pltpu.sync_copy(wA.at[pl.ds(0, nrows)],
                            mde_hbm.at[pl.ds(off, nrows)])
            pltpu.sync_copy(wC.at[pl.ds(0, nrows)],
                            s_hbm.at[pl.ds(off, nrows)])

        @pl.when(tid < NSUB - 1)
        def _():
            _self(ROWS_A)

        @pl.when(tid == NSUB - 1)
        def _():
            _self(N - (NSUB - 1) * ROWS_A)
            npad = ET - PAD_OFF

            def _gp(k, _):
                wA[pl.ds(k * 16, 16)] = jnp.full((16,), SENT, jnp.int32)
                wC[pl.ds(k * 16, 16)] = jnp.zeros((16,), jnp.float32)
                return 0
            lax.fori_loop(0, WB // 16, _gp, 0)
            done = 0
            while done < npad:
                n = min(WB, npad - done)
                pltpu.sync_copy(wA.at[pl.ds(0, n)],
                                se_hbm.at[pl.ds(PAD_OFF + done, n)])
                pltpu.sync_copy(wC.at[pl.ds(0, n)],
                                s_hbm.at[pl.ds(PAD_OFF + done, n)])
                done += n

            def _gq(k, _):
                wA[pl.ds(k * 16, 16)] = jnp.full((16,), N - 1, jnp.int32)
                return 0
            lax.fori_loop(0, WB // 16, _gq, 0)
            done = 0
            while done < npad:
                n = min(WB, npad - done)
                pltpu.sync_copy(wA.at[pl.ds(0, n)],
                                mse_hbm.at[pl.ds(PAD_OFF + done, n)])
                pltpu.sync_copy(wA.at[pl.ds(0, n)],
                                mde_hbm.at[pl.ds(PAD_OFF + done, n)])
                done += n


def _sc_scalers(src_p, dst_p):
    mesh = plsc.VectorSubcoreMesh(core_axis_name="c", subcore_axis_name="s",
                                  num_cores=NCORE, num_subcores=NSUB)
    kern = pl.kernel(
        _sca_body,
        out_type=[
            jax.ShapeDtypeStruct((ET,), jnp.int32),
            jax.ShapeDtypeStruct((ET,), jnp.int32),
            jax.ShapeDtypeStruct((ET,), jnp.int32),
            jax.ShapeDtypeStruct((ET,), jnp.float32),
        ],
        mesh=mesh,
        compiler_params=pltpu.CompilerParams(use_tc_tiling_on_sc=False,
                                             needs_layout_passes=False),
        scratch_types=[
            pltpu.VMEM((EPT1,), jnp.int32),
            pltpu.VMEM((EPT1,), jnp.int32),
            pltpu.VMEM((HS,), jnp.float32),
            pltpu.VMEM((WB,), jnp.int32),
            pltpu.VMEM((WB,), jnp.float32),
            pltpu.VMEM((HB,), jnp.float32),
            pltpu.VMEM((NSUB, HB), jnp.float32),
            pltpu.VMEM_SHARED((NSUB, HS), jnp.float32),
            pltpu.VMEM_SHARED((HS,), jnp.float32),
        ],
    )
    return kern(src_p, dst_p)


def _scb_body(se_hbm, mse_hbm, mde_hbm, s_hbm, stab_hbm, dtab_hbm,
              out_hbm,
              seS, mseS, mdeS, sS,
              srows, drows, msgB, idxS, idxD, sidx, seb, sb,
              acc, semGS, semGD, semW, semT):
    tid = lax.axis_index("s")
    cid = lax.axis_index("c")
    zero16 = jnp.zeros((16,), jnp.float32)
    base = tid * TPT

    def _zero_msg():
        def _zm(i, _):
            r = i // (Q // 16)
            c = (i % (Q // 16)) * 16
            msgB[0][r, pl.ds(c, 16)] = zero16
            return 0
        lax.fori_loop(0, C * (Q // 16), _zm, 0)

    def _zero_acc():
        def _za(nrows):
            off = 0
            while off < nrows:
                n = min(C, nrows - off)
                pltpu.sync_copy(msgB[0].at[pl.ds(0, n)],
                                acc.at[pl.ds(tid * ROWS_A + off, n)])
                off += n

        @pl.when(tid < NSUB - 1)
        def _():
            _za(ROWS_A)

        @pl.when(tid == NSUB - 1)
        def _():
            _za(ACC_R - (NSUB - 1) * ROWS_A)

    def _stage_fire(bi, par):
        boff = bi * (BS * C)
        pltpu.async_copy(se_hbm.at[pl.ds(base + boff, BS * C)], seS[par],
                         semT[par])
        pltpu.async_copy(mse_hbm.at[pl.ds(base + boff, BS * C)],
                         mseS[par], semT[par])
        pltpu.async_copy(mde_hbm.at[pl.ds(base + boff, BS * C)],
                         mdeS[par], semT[par])
        pltpu.async_copy(s_hbm.at[pl.ds(base + boff, BS * C)], sS[par],
                         semT[par])

    def _stage_wait(bi, par):
        boff = bi * (BS * C)
        pltpu.make_async_copy(se_hbm.at[pl.ds(base + boff, BS * C)],
                              seS[par], semT[par]).wait()
        pltpu.make_async_copy(mse_hbm.at[pl.ds(base + boff, BS * C)],
                              mseS[par], semT[par]).wait()
        pltpu.make_async_copy(mde_hbm.at[pl.ds(base + boff, BS * C)],
                              mdeS[par], semT[par]).wait()
        pltpu.make_async_copy(s_hbm.at[pl.ds(base + boff, BS * C)],
                              sS[par], semT[par]).wait()

    def _run_pass(p):
        qoff = (cid * 2 + p) * N

        def _fill_g(o, par, buf):
            def _f(k, _):
                j = o + k * 16
                idxS[buf][pl.ds(k * 16, 16)] = (
                    qoff + mseS[par][pl.ds(j, 16)])
                idxD[buf][pl.ds(k * 16, 16)] = (
                    qoff + mdeS[par][pl.ds(j, 16)])
                seb[buf][pl.ds(k * 16, 16)] = seS[par][pl.ds(j, 16)]
                sb[buf][pl.ds(k * 16, 16)] = sS[par][pl.ds(j, 16)]
                return 0
            lax.fori_loop(0, C // 16, _f, 0)

        def _fire_g(buf):
            pltpu.async_copy(stab_hbm.at[idxS[buf]], srows[buf],
                             semGS[buf])
            pltpu.async_copy(dtab_hbm.at[idxD[buf]], drows[buf],
                             semGD[buf])

        def _wait_g(buf):
            pltpu.make_async_copy(stab_hbm.at[idxS[buf]], srows[buf],
                                  semGS[buf]).wait()
            pltpu.make_async_copy(dtab_hbm.at[idxD[buf]], drows[buf],
                                  semGD[buf]).wait()

        def _wait_w(buf):
            pltpu.make_async_copy(msgB[buf], acc.at[sidx[buf]],
                                  semW[buf]).wait()

        def _compute(gbuf, mbuf):
            ilv = plsc.PackFormat.INTERLEAVED

            def _one(e):
                s = plsc.load_gather(
                    sb[gbuf], [jnp.full((16,), e, jnp.int32)])
                for v in range(Q // 32):
                    a2 = srows[gbuf][e, pl.ds(v * 32, 32)]
                    hd2 = srows[gbuf][e, pl.ds(Q + v * 32, 32)]
                    b2 = drows[gbuf][e, pl.ds(v * 32, 32)]
                    he2 = drows[gbuf][e, pl.ds(Q + v * 32, 32)]
                    xw2 = drows[gbuf][e, pl.ds(2 * Q + v * 32, 32)]
                    av = plsc.unpack(a2, format=ilv)
                    hdv = plsc.unpack(hd2, format=ilv)
                    bv = plsc.unpack(b2, format=ilv)
                    hev = plsc.unpack(he2, format=ilv)
                    xwv = plsc.unpack(xw2, format=ilv)
                    for h in range(2):
                        g = (jnp.maximum(av[h] + bv[h], 0.0)
                             + hdv[h] + hev[h])
                        msgB[mbuf][e, pl.ds((2 * v + h) * 16, 16)] = (
                            xwv[h] * (s * g))

            def _pe(j, _):
                _one(2 * j)
                _one(2 * j + 1)
                return 0
            lax.fori_loop(0, C // 2, _pe, 0)

        def _fill_sidx(gbuf, mbuf):
            def _f(k, _):
                sidx[mbuf][pl.ds(k * 16, 16)] = seb[gbuf][pl.ds(k * 16,
                                                                16)]
                return 0
            lax.fori_loop(0, C // 16, _f, 0)

        def _fire_w(buf):
            pltpu.async_copy(msgB[buf], acc.at[sidx[buf]], semW[buf],
                             add=True)

        _stage_fire(0, 0)
        _stage_wait(0, 0)
        _stage_fire(1, 1)
        for c in range(3):
            _fill_g(c * C, 0, c)
            _fire_g(c)

        def _block(b, _):
            for par in range(2):
                @pl.when(jnp.logical_and(b > 0, b % 2 == par))
                def _():
                    _stage_wait(b, par)

            for k in range(BS):
                gbuf = k % 3
                mbuf = k % 2
                _wait_g(gbuf)
                if k < 2:
                    @pl.when(b > 0)
                    def _():
                        _wait_w(mbuf)
                else:
                    _wait_w(mbuf)
                _compute(gbuf, mbuf)
                _fill_sidx(gbuf, mbuf)
                _fire_w(mbuf)
                if k + 3 < BS:
                    for par in range(2):
                        @pl.when(b % 2 == par)
                        def _():
                            _fill_g((k + 3) * C, par, gbuf)
                            _fire_g(gbuf)
                else:
                    for par in range(2):
                        @pl.when(jnp.logical_and(b + 1 < NBLK,
                                                 (b + 1) % 2 == par))
                        def _():
                            _fill_g((k + 3 - BS) * C, par, gbuf)
                            _fire_g(gbuf)

            for par in range(2):
                @pl.when(jnp.logical_and(b + 2 < NBLK, b % 2 == par))
                def _():
                    _stage_fire(b + 2, par)
            return 0
        lax.fori_loop(0, NBLK, _block, 0)

        _wait_w(0)
        _wait_w(1)
        plsc.subcore_barrier()

        def _co(nrows):
            off = 0
            while off < nrows:
                n = min(C, nrows - off)
                pltpu.sync_copy(acc.at[pl.ds(tid * ROWS_A + off, n)],
                                msgB[0].at[pl.ds(0, n)])
                pltpu.sync_copy(
                    msgB[0].at[pl.ds(0, n)],
                    out_hbm.at[pl.ds(qoff + tid * ROWS_A + off, n)])
                off += n

        @pl.when(tid < NSUB - 1)
        def _():
            _co(ROWS_A)

        @pl.when(tid == NSUB - 1)
        def _():
            _co(N - (NSUB - 1) * ROWS_A)
        plsc.subcore_barrier()

    for p in range(2):
        _zero_msg()
        _zero_acc()
        plsc.subcore_barrier()
        _run_pass(p)


def _sc_aggregate(se, mse, mde, s, stab4, dtab4):
    mesh = plsc.VectorSubcoreMesh(core_axis_name="c", subcore_axis_name="s",
                                  num_cores=NCORE, num_subcores=NSUB)
    kern = pl.kernel(
        _scb_body,
        out_type=jax.ShapeDtypeStruct((4 * N, Q), jnp.float32),
        mesh=mesh,
        compiler_params=pltpu.CompilerParams(use_tc_tiling_on_sc=False,
                                             needs_layout_passes=False),
        scratch_types=[
            [pltpu.VMEM((BS * C,), jnp.int32)] * 2,
            [pltpu.VMEM((BS * C,), jnp.int32)] * 2,
            [pltpu.VMEM((BS * C,), jnp.int32)] * 2,
            [pltpu.VMEM((BS * C,), jnp.float32)] * 2,
            [pltpu.VMEM((C, 2 * Q), jnp.bfloat16)] * 3,
            [pltpu.VMEM((C, 3 * Q), jnp.bfloat16)] * 3,
            [pltpu.VMEM((C, Q), jnp.float32)] * 2,
            [pltpu.VMEM((C,), jnp.int32)] * 3,
            [pltpu.VMEM((C,), jnp.int32)] * 3,
            [pltpu.VMEM((C,), jnp.int32)] * 2,
            [pltpu.VMEM((C,), jnp.int32)] * 3,
            [pltpu.VMEM((C,), jnp.float32)] * 3,
            pltpu.VMEM_SHARED((ACC_R, Q), jnp.float32),
            [pltpu.SemaphoreType.DMA] * 3,
            [pltpu.SemaphoreType.DMA] * 3,
            [pltpu.SemaphoreType.DMA] * 2,
            [pltpu.SemaphoreType.DMA] * 2,
        ],
    )
    return kern(se, mse, mde, s, stab4, dtab4)


def kernel(x, edge_index, W_w, W_b, U_w, U_b, V_w, V_b, D_w, D_b, E_w, E_b):
    d = x.shape[1]

    ut = U_w.T
    wt = W_w.T
    wcat = jnp.concatenate(
        [V_w[:, :d].T, V_w[:, d:].T, D_w.T, E_w.T], axis=1)
    bcat = jnp.concatenate(
        [jnp.zeros((d,), jnp.float32), V_b, D_b, E_b]).reshape(1, 4 * d)
    ub = U_b.reshape(1, d)
    wb = W_b.reshape(1, d)

    stab, dtab = _tc_tables(x, ut, ub, wcat, bcat, wt, wb)
    stab4 = stab.reshape(4 * N, 2 * Q).astype(jnp.bfloat16)
    dtab4 = dtab.reshape(4 * N, 3 * Q).astype(jnp.bfloat16)

    src_p = edge_index[0].astype(jnp.int32)
    dst_p = edge_index[1].astype(jnp.int32)

    se, mse, mde, s = _sc_scalers(src_p, dst_p)
    out4 = _sc_aggregate(se, mse, mde, s, stab4, dtab4)
    perm = [0] * Q
    for v in range(Q // 32):
        for h in range(2):
            for k in range(16):
                perm[(2 * v + h) * 16 + k] = 32 * v + 2 * k + h
    inv = [0] * Q
    for j, c in enumerate(perm):
        inv[c] = j
    out4 = out4[:, jnp.array(inv, dtype=jnp.int32)]
    out = out4.reshape(4, N, Q).transpose(1, 0, 2).reshape(N, d)
    return out
